# baseline (speedup 1.0000x reference)
import functools

import jax
import jax.numpy as jnp
from jax import lax
from jax.experimental import pallas as pl
from jax.experimental.pallas import tpu as pltpu
from jax.experimental.pallas import tpu_sc as plsc

N_NODES = 10000
N_EDGES = 320000
HIDDEN = 128
NCLS = 64
BATCH = 1024
ALPHA = 0.1
K_ITERS = 10

NCORE = 1
NSUB = 16
B_PER_TILE = BATCH // (NCORE * NSUB)
N_PAD = 10240
ROWS_PER_TILE = N_PAD // NSUB
CHUNK = 64
NBUF = 8
GLA = NBUF // 2
GRP = 16
NGRP = 20
EC_PER_TILE = GRP * NGRP
E_PAD = NSUB * EC_PER_TILE * CHUNK
UCH = 64
TBLK = 1024

_mesh = plsc.VectorSubcoreMesh(
    core_axis_name="c", subcore_axis_name="s", num_cores=NCORE,
    num_subcores=NSUB)
_sc_params = pltpu.CompilerParams(use_tc_tiling_on_sc=False)


def _deg_body(row_hbm, col_hbm, dout_hbm, din_hbm, rbuf, cbuf, ones, zbuf,
              do_sp, di_sp):
    s = lax.axis_index("s")
    for i in range(CHUNK // 16):
        ones[pl.ds(i * 16, 16)] = jnp.full((16,), 1.0, jnp.float32)
    for i in range(ROWS_PER_TILE // 16):
        zbuf[pl.ds(i * 16, 16)] = jnp.zeros((16,), jnp.float32)
    sl = pl.ds(s * ROWS_PER_TILE, ROWS_PER_TILE)
    pltpu.sync_copy(zbuf, do_sp.at[sl])
    pltpu.sync_copy(zbuf, di_sp.at[sl])
    pltpu.sync_copy(row_hbm.at[s], rbuf)
    pltpu.sync_copy(col_hbm.at[s], cbuf)
    plsc.subcore_barrier()

    def body(j, carry):
        pltpu.sync_copy(ones, do_sp.at[rbuf.at[j]], add=True)
        pltpu.sync_copy(ones, di_sp.at[cbuf.at[j]], add=True)
        return carry

    lax.fori_loop(0, EC_PER_TILE, body, 0)
    plsc.subcore_barrier()
    pltpu.sync_copy(do_sp.at[sl], dout_hbm.at[sl])
    pltpu.sync_copy(di_sp.at[sl], din_hbm.at[sl])


_deg_kernel = functools.partial(
    pl.kernel,
    out_type=(jax.ShapeDtypeStruct((N_PAD,), jnp.float32),
              jax.ShapeDtypeStruct((N_PAD,), jnp.float32)),
    mesh=_mesh,
    scratch_types=[
        pltpu.VMEM((EC_PER_TILE, CHUNK), jnp.int32),
        pltpu.VMEM((EC_PER_TILE, CHUNK), jnp.int32),
        pltpu.VMEM((CHUNK,), jnp.float32),
        pltpu.VMEM((ROWS_PER_TILE,), jnp.float32),
        pltpu.VMEM_SHARED((N_PAD,), jnp.float32),
        pltpu.VMEM_SHARED((N_PAD,), jnp.float32),
    ],
    compiler_params=_sc_params,
)(_deg_body)


def _prep_body(emb_ref, w_ref, dout_ref, din_ref,
               c1_ref, p16_ref, qy_ref):
    x = emb_ref[...]
    ss = jnp.sum(x * x, axis=-1, keepdims=True)
    h = x / (jnp.sqrt(ss) + 1e-12)
    y0 = jnp.dot(h, w_ref[...], preferred_element_type=jnp.float32)
    din = din_ref[...]
    dout = dout_ref[...]
    rin = lax.rsqrt(jnp.where(din > 0, din, 1.0))
    rout = lax.rsqrt(jnp.where(dout > 0, dout, 1.0))
    c1_ref[...] = ALPHA * rin * y0
    p16_ref[...] = jnp.broadcast_to((1.0 - ALPHA) * rin * rout, (TBLK, 16))
    qy_ref[...] = jnp.concatenate([
        ALPHA * y0,
        jnp.broadcast_to((1.0 - ALPHA) * rout, (TBLK, 16)),
        jnp.zeros((TBLK, 48), jnp.float32),
    ], axis=1)


def _prep(emb_pad, w, dout, din):
    grid = (N_PAD // TBLK,)
    return pl.pallas_call(
        _prep_body,
        grid=grid,
        in_specs=[
            pl.BlockSpec((TBLK, HIDDEN), lambda i: (i, 0)),
            pl.BlockSpec((HIDDEN, NCLS), lambda i: (0, 0)),
            pl.BlockSpec((TBLK, 1), lambda i: (i, 0)),
            pl.BlockSpec((TBLK, 1), lambda i: (i, 0)),
        ],
        out_specs=[
            pl.BlockSpec((TBLK, NCLS), lambda i: (i, 0)),
            pl.BlockSpec((TBLK, 16), lambda i: (i, 0)),
            pl.BlockSpec((TBLK, 128), lambda i: (i, 0)),
        ],
        out_shape=[
            jax.ShapeDtypeStruct((N_PAD, NCLS), jnp.float32),
            jax.ShapeDtypeStruct((N_PAD, 16), jnp.float32),
            jax.ShapeDtypeStruct((N_PAD, 128), jnp.float32),
        ],
    )(emb_pad, w, dout, din)


def _main_body(row_hbm, col_hbm, c1_hbm, p16_hbm, qy_hbm,
               idx_hbm, b_hbm, out_hbm, u_hbm,
               rbufc, cbufc,
               gbuf0, gbuf1, gbuf2, gbuf3, gbuf4, gbuf5, gbuf6, gbuf7,
               abuf, ubuf, c1buf, p16buf,
               idxbuf, qybuf, rdbuf, obuf, bbuf,
               sem0, sem1, sem2, sem3, sem4, sem5, sem6, sem7,
               ssem0, ssem1, ssem2, ssem3, ssem4, ssem5, ssem6, ssem7,
               S_sp):
    s = lax.axis_index("s")
    rbase = s * ROWS_PER_TILE
    gb = (gbuf0, gbuf1, gbuf2, gbuf3, gbuf4, gbuf5, gbuf6, gbuf7)
    sm = (sem0, sem1, sem2, sem3, sem4, sem5, sem6, sem7)
    ssm = (ssem0, ssem1, ssem2, ssem3, ssem4, ssem5, ssem6, ssem7)

    def zb(i, carry):
        for cc in range(NCLS // 16):
            obuf[i, pl.ds(cc * 16, 16)] = jnp.zeros((16,), jnp.float32)
        return carry

    lax.fori_loop(0, UCH, zb, 0)

    def init_chunk(t, carry):
        base = rbase + t * UCH
        pltpu.sync_copy(c1_hbm.at[pl.ds(base, UCH)], c1buf)

        def rw(i, carry2):
            for cc in range(NCLS // 16):
                sl = pl.ds(cc * 16, 16)
                ubuf[i, sl] = c1buf[i, sl] * (1.0 / ALPHA)
            return carry2

        lax.fori_loop(0, UCH, rw, 0)
        pltpu.sync_copy(ubuf, u_hbm.at[pl.ds(base, UCH)])
        pltpu.sync_copy(obuf, S_sp.at[pl.ds(base, UCH)])
        return carry

    lax.fori_loop(0, ROWS_PER_TILE // UCH, init_chunk, 0)
    plsc.subcore_barrier()

    def scatter_phase():
        def group(g, carry):
            pltpu.sync_copy(row_hbm.at[s, pl.ds(g * GRP, GRP)], rbufc)
            pltpu.sync_copy(col_hbm.at[s, pl.ds(g * GRP, GRP)], cbufc)
            gd = [None] * NBUF
            sd = [None] * NBUF
            for j in range(GLA):
                gd[j] = pltpu.async_copy(
                    u_hbm.at[cbufc.at[j]], gb[j], sm[j])
            for j in range(GRP):
                cur = j % NBUF
                gd[cur].wait()
                sd[cur] = pltpu.async_copy(
                    gb[cur], S_sp.at[rbufc.at[j]], ssm[cur], add=True)
                if j + GLA < GRP:
                    nx = (j + GLA) % NBUF
                    if sd[nx] is not None:
                        sd[nx].wait()
                    gd[nx] = pltpu.async_copy(
                        u_hbm.at[cbufc.at[j + GLA]], gb[nx], sm[nx])
            for j in range(GRP - NBUF, GRP):
                sd[j % NBUF].wait()
            return carry

        lax.fori_loop(0, NGRP, group, 0)

    def update_phase():
        def uchunk(t, carry):
            base = rbase + t * UCH
            pltpu.sync_copy(S_sp.at[pl.ds(base, UCH)], abuf)
            pltpu.sync_copy(c1_hbm.at[pl.ds(base, UCH)], c1buf)
            pltpu.sync_copy(p16_hbm.at[pl.ds(base, UCH)], p16buf)

            def rw(i, carry2):
                p = p16buf[i]
                for cc in range(NCLS // 16):
                    sl = pl.ds(cc * 16, 16)
                    ubuf[i, sl] = p * abuf[i, sl] + c1buf[i, sl]
                return carry2

            lax.fori_loop(0, UCH, rw, 0)
            pltpu.sync_copy(ubuf, u_hbm.at[pl.ds(base, UCH)])
            pltpu.sync_copy(obuf, S_sp.at[pl.ds(base, UCH)])
            return carry

        lax.fori_loop(0, ROWS_PER_TILE // UCH, uchunk, 0)

    def kiter(k, carry):
        scatter_phase()
        plsc.subcore_barrier()
        update_phase()
        plsc.subcore_barrier()
        return carry

    lax.fori_loop(0, K_ITERS - 1, kiter, 0)
    scatter_phase()
    plsc.subcore_barrier()

    pltpu.sync_copy(b_hbm, bbuf)
    for h in range(B_PER_TILE // 16):
        ob = s * B_PER_TILE + h * 16
        pltpu.sync_copy(idx_hbm.at[pl.ds(ob, 16)], idxbuf)
        pltpu.sync_copy(qy_hbm.at[idxbuf], qybuf)
        pltpu.sync_copy(S_sp.at[idxbuf], rdbuf)

        def rbody(i, carry):
            q = qybuf[i, pl.ds(NCLS, 16)]
            for cc in range(NCLS // 16):
                sl = pl.ds(cc * 16, 16)
                obuf[h * 16 + i, sl] = (q * rdbuf[i, sl] + qybuf[i, sl]
                                        + bbuf[sl])
            return carry

        lax.fori_loop(0, 16, rbody, 0)
    pltpu.sync_copy(obuf, out_hbm.at[pl.ds(s * B_PER_TILE, B_PER_TILE)])


_main_kernel = functools.partial(
    pl.kernel,
    out_type=(jax.ShapeDtypeStruct((BATCH, NCLS), jnp.float32),
              jax.ShapeDtypeStruct((N_PAD, NCLS), jnp.float32)),
    mesh=_mesh,
    scratch_types=[
        pltpu.VMEM((GRP, CHUNK), jnp.int32),
        pltpu.VMEM((GRP, CHUNK), jnp.int32),
        pltpu.VMEM((CHUNK, NCLS), jnp.float32),
        pltpu.VMEM((CHUNK, NCLS), jnp.float32),
        pltpu.VMEM((CHUNK, NCLS), jnp.float32),
        pltpu.VMEM((CHUNK, NCLS), jnp.float32),
        pltpu.VMEM((CHUNK, NCLS), jnp.float32),
        pltpu.VMEM((CHUNK, NCLS), jnp.float32),
        pltpu.VMEM((CHUNK, NCLS), jnp.float32),
        pltpu.VMEM((CHUNK, NCLS), jnp.float32),
        pltpu.VMEM((UCH, NCLS), jnp.float32),
        pltpu.VMEM((UCH, NCLS), jnp.float32),
        pltpu.VMEM((UCH, NCLS), jnp.float32),
        pltpu.VMEM((UCH, 16), jnp.float32),
        pltpu.VMEM((16,), jnp.int32),
        pltpu.VMEM((16, 128), jnp.float32),
        pltpu.VMEM((16, NCLS), jnp.float32),
        pltpu.VMEM((UCH, NCLS), jnp.float32),
        pltpu.VMEM((NCLS,), jnp.float32),
        pltpu.SemaphoreType.DMA,
        pltpu.SemaphoreType.DMA,
        pltpu.SemaphoreType.DMA,
        pltpu.SemaphoreType.DMA,
        pltpu.SemaphoreType.DMA,
        pltpu.SemaphoreType.DMA,
        pltpu.SemaphoreType.DMA,
        pltpu.SemaphoreType.DMA,
        pltpu.SemaphoreType.DMA,
        pltpu.SemaphoreType.DMA,
        pltpu.SemaphoreType.DMA,
        pltpu.SemaphoreType.DMA,
        pltpu.SemaphoreType.DMA,
        pltpu.SemaphoreType.DMA,
        pltpu.SemaphoreType.DMA,
        pltpu.SemaphoreType.DMA,
        pltpu.VMEM_SHARED((N_PAD, NCLS), jnp.float32),
    ],
    compiler_params=_sc_params,
)(_main_body)


def kernel(X, idx, edge_index, emb, W, b):
    del X
    emb_pad = jnp.pad(emb, ((0, N_PAD - N_NODES), (0, 0)))
    row = edge_index[0].astype(jnp.int32)
    col = edge_index[1].astype(jnp.int32)
    padv = jnp.full((E_PAD - N_EDGES,), N_NODES, jnp.int32)
    row3 = jnp.concatenate([row, padv]).reshape(NSUB, EC_PER_TILE, CHUNK)
    col3 = jnp.concatenate([col, padv]).reshape(NSUB, EC_PER_TILE, CHUNK)
    idx32 = idx.astype(jnp.int32)

    dout, din = _deg_kernel(row3, col3)
    c1, p16, qy = _prep(emb_pad, W, dout.reshape(N_PAD, 1),
                        din.reshape(N_PAD, 1))
    out, _ = _main_kernel(row3, col3, c1, p16, qy, idx32, b)
    return out


# speedup vs baseline: 1.7401x; 1.7401x over previous
import functools

import jax
import jax.numpy as jnp
from jax import lax
from jax.experimental import pallas as pl
from jax.experimental.pallas import tpu as pltpu
from jax.experimental.pallas import tpu_sc as plsc

N_NODES = 10000
N_EDGES = 320000
HIDDEN = 128
NCLS = 64
BATCH = 1024
ALPHA = 0.1
K_ITERS = 10

NCORE = 1
NSUB = 16
B_PER_TILE = BATCH // (NCORE * NSUB)
N_PAD = 10240
ROWS_PER_TILE = N_PAD // NSUB
CHUNK = 64
NBUF = 4
GLA = NBUF // 2
GRP = 16
IGRP = 4
NGRP = 20
EC_PER_TILE = GRP * NGRP
E_PAD = NSUB * EC_PER_TILE * CHUNK
UCH = 64
TBLK = 1024

_mesh = plsc.VectorSubcoreMesh(
    core_axis_name="c", subcore_axis_name="s", num_cores=NCORE,
    num_subcores=NSUB)
_sc_params = pltpu.CompilerParams(use_tc_tiling_on_sc=False)


def _deg_body(row_hbm, col_hbm, dout_hbm, din_hbm, rbuf, cbuf, ones, zbuf,
              do_sp, di_sp):
    s = lax.axis_index("s")
    for i in range(CHUNK // 16):
        ones[pl.ds(i * 16, 16)] = jnp.full((16,), 1.0, jnp.float32)
    for i in range(ROWS_PER_TILE // 16):
        zbuf[pl.ds(i * 16, 16)] = jnp.zeros((16,), jnp.float32)
    sl = pl.ds(s * ROWS_PER_TILE, ROWS_PER_TILE)
    pltpu.sync_copy(zbuf, do_sp.at[sl])
    pltpu.sync_copy(zbuf, di_sp.at[sl])
    pltpu.sync_copy(row_hbm.at[s], rbuf)
    pltpu.sync_copy(col_hbm.at[s], cbuf)
    plsc.subcore_barrier()

    def body(j, carry):
        pltpu.sync_copy(ones, do_sp.at[rbuf.at[j]], add=True)
        pltpu.sync_copy(ones, di_sp.at[cbuf.at[j]], add=True)
        return carry

    lax.fori_loop(0, EC_PER_TILE, body, 0)
    plsc.subcore_barrier()
    pltpu.sync_copy(do_sp.at[sl], dout_hbm.at[sl])
    pltpu.sync_copy(di_sp.at[sl], din_hbm.at[sl])


_deg_kernel = functools.partial(
    pl.kernel,
    out_type=(jax.ShapeDtypeStruct((N_PAD,), jnp.float32),
              jax.ShapeDtypeStruct((N_PAD,), jnp.float32)),
    mesh=_mesh,
    scratch_types=[
        pltpu.VMEM((EC_PER_TILE, CHUNK), jnp.int32),
        pltpu.VMEM((EC_PER_TILE, CHUNK), jnp.int32),
        pltpu.VMEM((CHUNK,), jnp.float32),
        pltpu.VMEM((ROWS_PER_TILE,), jnp.float32),
        pltpu.VMEM_SHARED((N_PAD,), jnp.float32),
        pltpu.VMEM_SHARED((N_PAD,), jnp.float32),
    ],
    compiler_params=_sc_params,
)(_deg_body)


def _prep_body(emb_ref, w_ref, dout_ref, din_ref,
               c1_ref, p16_ref, qy_ref):
    x = emb_ref[...]
    ss = jnp.sum(x * x, axis=-1, keepdims=True)
    h = x / (jnp.sqrt(ss) + 1e-12)
    y0 = jnp.dot(h, w_ref[...], preferred_element_type=jnp.float32)
    din = din_ref[...]
    dout = dout_ref[...]
    rin = lax.rsqrt(jnp.where(din > 0, din, 1.0))
    rout = lax.rsqrt(jnp.where(dout > 0, dout, 1.0))
    c1_ref[...] = ALPHA * rin * y0
    p16_ref[...] = jnp.broadcast_to((1.0 - ALPHA) * rin * rout, (TBLK, 16))
    qy_ref[...] = jnp.concatenate([
        ALPHA * y0,
        jnp.broadcast_to((1.0 - ALPHA) * rout, (TBLK, 16)),
        jnp.zeros((TBLK, 48), jnp.float32),
    ], axis=1)


def _prep(emb_pad, w, dout, din):
    grid = (N_PAD // TBLK,)
    return pl.pallas_call(
        _prep_body,
        grid=grid,
        in_specs=[
            pl.BlockSpec((TBLK, HIDDEN), lambda i: (i, 0)),
            pl.BlockSpec((HIDDEN, NCLS), lambda i: (0, 0)),
            pl.BlockSpec((TBLK, 1), lambda i: (i, 0)),
            pl.BlockSpec((TBLK, 1), lambda i: (i, 0)),
        ],
        out_specs=[
            pl.BlockSpec((TBLK, NCLS), lambda i: (i, 0)),
            pl.BlockSpec((TBLK, 16), lambda i: (i, 0)),
            pl.BlockSpec((TBLK, 128), lambda i: (i, 0)),
        ],
        out_shape=[
            jax.ShapeDtypeStruct((N_PAD, NCLS), jnp.float32),
            jax.ShapeDtypeStruct((N_PAD, 16), jnp.float32),
            jax.ShapeDtypeStruct((N_PAD, 128), jnp.float32),
        ],
    )(emb_pad, w, dout, din)


def _main_body(row_hbm, col_hbm, c1_hbm, p16_hbm, qy_hbm,
               idx_hbm, b_hbm, out_hbm,
               rbufc, cbufc,
               gbuf0, gbuf1, gbuf2, gbuf3,
               abuf, ubuf, c1buf, p16buf,
               idxbuf, qybuf, rdbuf, obuf, bbuf,
               sem0, sem1, sem2, sem3,
               ssem0, ssem1, ssem2, ssem3,
               U_sp, S_sp):
    s = lax.axis_index("s")
    rbase = s * ROWS_PER_TILE
    gb = (gbuf0, gbuf1, gbuf2, gbuf3)
    sm = (sem0, sem1, sem2, sem3)
    ssm = (ssem0, ssem1, ssem2, ssem3)

    def zb(i, carry):
        for cc in range(NCLS // 16):
            obuf[i, pl.ds(cc * 16, 16)] = jnp.zeros((16,), jnp.float32)
        return carry

    lax.fori_loop(0, UCH, zb, 0)

    def init_chunk(t, carry):
        base = rbase + t * UCH
        pltpu.sync_copy(c1_hbm.at[pl.ds(base, UCH)], c1buf)

        def rw(i, carry2):
            for cc in range(NCLS // 16):
                sl = pl.ds(cc * 16, 16)
                ubuf[i, sl] = c1buf[i, sl] * (1.0 / ALPHA)
            return carry2

        lax.fori_loop(0, UCH, rw, 0)
        pltpu.sync_copy(ubuf, U_sp.at[pl.ds(base, UCH)])
        pltpu.sync_copy(obuf, S_sp.at[pl.ds(base, UCH)])
        return carry

    lax.fori_loop(0, ROWS_PER_TILE // UCH, init_chunk, 0)
    plsc.subcore_barrier()

    def scatter_phase():
        def sgroup(g, carry):
            pltpu.sync_copy(
                row_hbm.at[s, pl.ds(g * IGRP * GRP, IGRP * GRP)], rbufc)
            pltpu.sync_copy(
                col_hbm.at[s, pl.ds(g * IGRP * GRP, IGRP * GRP)], cbufc)
            for sub in range(IGRP):
                o = sub * GRP
                gd = [None] * NBUF
                sd = [None] * NBUF
                for j in range(GLA):
                    gd[j] = pltpu.async_copy(
                        U_sp.at[cbufc.at[o + j]], gb[j], sm[j])
                for j in range(GRP):
                    cur = j % NBUF
                    gd[cur].wait()
                    sd[cur] = pltpu.async_copy(
                        gb[cur], S_sp.at[rbufc.at[o + j]], ssm[cur],
                        add=True)
                    if j + GLA < GRP:
                        nx = (j + GLA) % NBUF
                        if sd[nx] is not None:
                            sd[nx].wait()
                        gd[nx] = pltpu.async_copy(
                            U_sp.at[cbufc.at[o + j + GLA]], gb[nx], sm[nx])
                for j in range(GRP - NBUF, GRP):
                    sd[j % NBUF].wait()
            return carry

        lax.fori_loop(0, NGRP // IGRP, sgroup, 0)

    def update_phase():
        def uchunk(t, carry):
            base = rbase + t * UCH
            pltpu.sync_copy(S_sp.at[pl.ds(base, UCH)], abuf)
            pltpu.sync_copy(c1_hbm.at[pl.ds(base, UCH)], c1buf)
            pltpu.sync_copy(p16_hbm.at[pl.ds(base, UCH)], p16buf)

            def rw(i, carry2):
                p = p16buf[i]
                for cc in range(NCLS // 16):
                    sl = pl.ds(cc * 16, 16)
                    ubuf[i, sl] = p * abuf[i, sl] + c1buf[i, sl]
                return carry2

            lax.fori_loop(0, UCH, rw, 0)
            pltpu.sync_copy(ubuf, U_sp.at[pl.ds(base, UCH)])
            pltpu.sync_copy(obuf, S_sp.at[pl.ds(base, UCH)])
            return carry

        lax.fori_loop(0, ROWS_PER_TILE // UCH, uchunk, 0)

    def kiter(k, carry):
        scatter_phase()
        plsc.subcore_barrier()
        update_phase()
        plsc.subcore_barrier()
        return carry

    lax.fori_loop(0, K_ITERS - 1, kiter, 0)
    scatter_phase()
    plsc.subcore_barrier()

    pltpu.sync_copy(b_hbm, bbuf)
    for h in range(B_PER_TILE // 16):
        ob = s * B_PER_TILE + h * 16
        pltpu.sync_copy(idx_hbm.at[pl.ds(ob, 16)], idxbuf)
        pltpu.sync_copy(qy_hbm.at[idxbuf], qybuf)
        pltpu.sync_copy(S_sp.at[idxbuf], rdbuf)

        def rbody(i, carry):
            q = qybuf[i, pl.ds(NCLS, 16)]
            for cc in range(NCLS // 16):
                sl = pl.ds(cc * 16, 16)
                obuf[h * 16 + i, sl] = (q * rdbuf[i, sl] + qybuf[i, sl]
                                        + bbuf[sl])
            return carry

        lax.fori_loop(0, 16, rbody, 0)
    pltpu.sync_copy(obuf, out_hbm.at[pl.ds(s * B_PER_TILE, B_PER_TILE)])


_main_kernel = functools.partial(
    pl.kernel,
    out_type=jax.ShapeDtypeStruct((BATCH, NCLS), jnp.float32),
    mesh=_mesh,
    scratch_types=[
        pltpu.VMEM((IGRP * GRP, CHUNK), jnp.int32),
        pltpu.VMEM((IGRP * GRP, CHUNK), jnp.int32),
        pltpu.VMEM((CHUNK, NCLS), jnp.float32),
        pltpu.VMEM((CHUNK, NCLS), jnp.float32),
        pltpu.VMEM((CHUNK, NCLS), jnp.float32),
        pltpu.VMEM((CHUNK, NCLS), jnp.float32),
        pltpu.VMEM((UCH, NCLS), jnp.float32),
        pltpu.VMEM((UCH, NCLS), jnp.float32),
        pltpu.VMEM((UCH, NCLS), jnp.float32),
        pltpu.VMEM((UCH, 16), jnp.float32),
        pltpu.VMEM((16,), jnp.int32),
        pltpu.VMEM((16, 128), jnp.float32),
        pltpu.VMEM((16, NCLS), jnp.float32),
        pltpu.VMEM((UCH, NCLS), jnp.float32),
        pltpu.VMEM((NCLS,), jnp.float32),
        pltpu.SemaphoreType.DMA,
        pltpu.SemaphoreType.DMA,
        pltpu.SemaphoreType.DMA,
        pltpu.SemaphoreType.DMA,
        pltpu.SemaphoreType.DMA,
        pltpu.SemaphoreType.DMA,
        pltpu.SemaphoreType.DMA,
        pltpu.SemaphoreType.DMA,
        pltpu.VMEM_SHARED((N_PAD, NCLS), jnp.float32),
        pltpu.VMEM_SHARED((N_PAD, NCLS), jnp.float32),
    ],
    compiler_params=_sc_params,
)(_main_body)


def kernel(X, idx, edge_index, emb, W, b):
    del X
    emb_pad = jnp.pad(emb, ((0, N_PAD - N_NODES), (0, 0)))
    row = edge_index[0].astype(jnp.int32)
    col = edge_index[1].astype(jnp.int32)
    padv = jnp.full((E_PAD - N_EDGES,), N_NODES, jnp.int32)
    row3 = jnp.concatenate([row, padv]).reshape(NSUB, EC_PER_TILE, CHUNK)
    col3 = jnp.concatenate([col, padv]).reshape(NSUB, EC_PER_TILE, CHUNK)
    idx32 = idx.astype(jnp.int32)

    dout, din = _deg_kernel(row3, col3)
    c1, p16, qy = _prep(emb_pad, W, dout.reshape(N_PAD, 1),
                        din.reshape(N_PAD, 1))
    out = _main_kernel(row3, col3, c1, p16, qy, idx32, b)
    return out


# speedup vs baseline: 1.8491x; 1.0626x over previous
import functools

import jax
import jax.numpy as jnp
from jax import lax
from jax.experimental import pallas as pl
from jax.experimental.pallas import tpu as pltpu
from jax.experimental.pallas import tpu_sc as plsc

N_NODES = 10000
N_EDGES = 320000
HIDDEN = 128
NCLS = 64
BATCH = 1024
ALPHA = 0.1
K_ITERS = 10

NCORE = 1
NSUB = 16
B_PER_TILE = BATCH // (NCORE * NSUB)
N_PAD = 10240
ROWS_PER_TILE = N_PAD // NSUB
CHUNK = 64
NBUF = 4
GLA = NBUF // 2
GRP = 16
IGRP = 4
NGRP = 20
EC_PER_TILE = GRP * NGRP
E_PAD = NSUB * EC_PER_TILE * CHUNK
UCH = 32
NUCH = ROWS_PER_TILE // UCH
TBLK = 1024

_mesh = plsc.VectorSubcoreMesh(
    core_axis_name="c", subcore_axis_name="s", num_cores=NCORE,
    num_subcores=NSUB)
_sc_params = pltpu.CompilerParams(use_tc_tiling_on_sc=False)


def _deg_body(row_hbm, col_hbm, dout_hbm, din_hbm, rbuf, cbuf, ones, zbuf,
              do_sp, di_sp):
    s = lax.axis_index("s")
    for i in range(CHUNK // 16):
        ones[pl.ds(i * 16, 16)] = jnp.full((16,), 1.0, jnp.float32)
    for i in range(ROWS_PER_TILE // 16):
        zbuf[pl.ds(i * 16, 16)] = jnp.zeros((16,), jnp.float32)
    sl = pl.ds(s * ROWS_PER_TILE, ROWS_PER_TILE)
    pltpu.sync_copy(zbuf, do_sp.at[sl])
    pltpu.sync_copy(zbuf, di_sp.at[sl])
    pltpu.sync_copy(row_hbm.at[s], rbuf)
    pltpu.sync_copy(col_hbm.at[s], cbuf)
    plsc.subcore_barrier()

    def body(j, carry):
        pltpu.sync_copy(ones, do_sp.at[rbuf.at[j]], add=True)
        pltpu.sync_copy(ones, di_sp.at[cbuf.at[j]], add=True)
        return carry

    lax.fori_loop(0, EC_PER_TILE, body, 0)
    plsc.subcore_barrier()
    pltpu.sync_copy(do_sp.at[sl], dout_hbm.at[sl])
    pltpu.sync_copy(di_sp.at[sl], din_hbm.at[sl])


_deg_kernel = functools.partial(
    pl.kernel,
    out_type=(jax.ShapeDtypeStruct((N_PAD,), jnp.float32),
              jax.ShapeDtypeStruct((N_PAD,), jnp.float32)),
    mesh=_mesh,
    scratch_types=[
        pltpu.VMEM((EC_PER_TILE, CHUNK), jnp.int32),
        pltpu.VMEM((EC_PER_TILE, CHUNK), jnp.int32),
        pltpu.VMEM((CHUNK,), jnp.float32),
        pltpu.VMEM((ROWS_PER_TILE,), jnp.float32),
        pltpu.VMEM_SHARED((N_PAD,), jnp.float32),
        pltpu.VMEM_SHARED((N_PAD,), jnp.float32),
    ],
    compiler_params=_sc_params,
)(_deg_body)


def _prep_body(emb_ref, w_ref, dout_ref, din_ref,
               c1_ref, p16_ref, qy_ref):
    x = emb_ref[...]
    ss = jnp.sum(x * x, axis=-1, keepdims=True)
    h = x / (jnp.sqrt(ss) + 1e-12)
    y0 = jnp.dot(h, w_ref[...], preferred_element_type=jnp.float32)
    din = din_ref[...]
    dout = dout_ref[...]
    rin = lax.rsqrt(jnp.where(din > 0, din, 1.0))
    rout = lax.rsqrt(jnp.where(dout > 0, dout, 1.0))
    c1_ref[...] = ALPHA * rin * y0
    p16_ref[...] = jnp.broadcast_to((1.0 - ALPHA) * rin * rout, (TBLK, 16))
    qy_ref[...] = jnp.concatenate([
        ALPHA * y0,
        jnp.broadcast_to((1.0 - ALPHA) * rout, (TBLK, 16)),
        jnp.zeros((TBLK, 48), jnp.float32),
    ], axis=1)


def _prep(emb_pad, w, dout, din):
    grid = (N_PAD // TBLK,)
    return pl.pallas_call(
        _prep_body,
        grid=grid,
        in_specs=[
            pl.BlockSpec((TBLK, HIDDEN), lambda i: (i, 0)),
            pl.BlockSpec((HIDDEN, NCLS), lambda i: (0, 0)),
            pl.BlockSpec((TBLK, 1), lambda i: (i, 0)),
            pl.BlockSpec((TBLK, 1), lambda i: (i, 0)),
        ],
        out_specs=[
            pl.BlockSpec((TBLK, NCLS), lambda i: (i, 0)),
            pl.BlockSpec((TBLK, 16), lambda i: (i, 0)),
            pl.BlockSpec((TBLK, 128), lambda i: (i, 0)),
        ],
        out_shape=[
            jax.ShapeDtypeStruct((N_PAD, NCLS), jnp.float32),
            jax.ShapeDtypeStruct((N_PAD, 16), jnp.float32),
            jax.ShapeDtypeStruct((N_PAD, 128), jnp.float32),
        ],
    )(emb_pad, w, dout, din)


def _main_body(row_hbm, col_hbm, c1_hbm, p16_hbm, qy_hbm,
               idx_hbm, b_hbm, out_hbm,
               rbufc, cbufc,
               gbuf0, gbuf1, gbuf2, gbuf3,
               abufA, abufB, ubufA, ubufB, c1bufA, c1bufB,
               p16bufA, p16bufB,
               idxbuf, qybuf, rdbuf, obuf, bbuf,
               sem0, sem1, sem2, sem3,
               ssem0, ssem1, ssem2, ssem3,
               usemSA, usemSB, usemCA, usemCB, usemPA, usemPB,
               U_sp, S_sp):
    s = lax.axis_index("s")
    rbase = s * ROWS_PER_TILE
    gb = (gbuf0, gbuf1, gbuf2, gbuf3)
    sm = (sem0, sem1, sem2, sem3)
    ssm = (ssem0, ssem1, ssem2, ssem3)
    ub = ((abufA, ubufA, c1bufA, p16bufA, usemSA, usemCA, usemPA),
          (abufB, ubufB, c1bufB, p16bufB, usemSB, usemCB, usemPB))

    def zb(i, carry):
        for cc in range(NCLS // 16):
            obuf[i, pl.ds(cc * 16, 16)] = jnp.zeros((16,), jnp.float32)
        return carry

    lax.fori_loop(0, B_PER_TILE, zb, 0)
    zsrc = obuf.at[pl.ds(0, UCH)]

    def init_chunk(t, carry):
        base = rbase + t * UCH
        pltpu.sync_copy(c1_hbm.at[pl.ds(base, UCH)], c1bufA)

        def rw(i, carry2):
            for cc in range(NCLS // 16):
                sl = pl.ds(cc * 16, 16)
                ubufA[i, sl] = c1bufA[i, sl] * (1.0 / ALPHA)
            return carry2

        lax.fori_loop(0, UCH, rw, 0)
        pltpu.sync_copy(ubufA, U_sp.at[pl.ds(base, UCH)])
        pltpu.sync_copy(zsrc, S_sp.at[pl.ds(base, UCH)])
        return carry

    lax.fori_loop(0, NUCH, init_chunk, 0)
    plsc.subcore_barrier()

    def scatter_phase():
        def sgroup(g, carry):
            pltpu.sync_copy(
                row_hbm.at[s, pl.ds(g * IGRP * GRP, IGRP * GRP)], rbufc)
            pltpu.sync_copy(
                col_hbm.at[s, pl.ds(g * IGRP * GRP, IGRP * GRP)], cbufc)
            for sub in range(IGRP):
                o = sub * GRP
                gd = [None] * NBUF
                sd = [None] * NBUF
                for j in range(GLA):
                    gd[j] = pltpu.async_copy(
                        U_sp.at[cbufc.at[o + j]], gb[j], sm[j])
                for j in range(GRP):
                    cur = j % NBUF
                    gd[cur].wait()
                    sd[cur] = pltpu.async_copy(
                        gb[cur], S_sp.at[rbufc.at[o + j]], ssm[cur],
                        add=True)
                    if j + GLA < GRP:
                        nx = (j + GLA) % NBUF
                        if sd[nx] is not None:
                            sd[nx].wait()
                        gd[nx] = pltpu.async_copy(
                            U_sp.at[cbufc.at[o + j + GLA]], gb[nx], sm[nx])
                for j in range(GRP - NBUF, GRP):
                    sd[j % NBUF].wait()
            return carry

        lax.fori_loop(0, NGRP // IGRP, sgroup, 0)

    def _uissue(t, bs):
        ab, _, cb, pb, sS, sC, sP = ub[bs]
        base = rbase + t * UCH
        pltpu.async_copy(S_sp.at[pl.ds(base, UCH)], ab, sS)
        pltpu.async_copy(c1_hbm.at[pl.ds(base, UCH)], cb, sC)
        pltpu.async_copy(p16_hbm.at[pl.ds(base, UCH)], pb, sP)

    def _ucompute(t, bs):
        ab, uu, cb, pb, sS, sC, sP = ub[bs]
        base = rbase + t * UCH
        pltpu.make_async_copy(S_sp.at[pl.ds(base, UCH)], ab, sS).wait()
        pltpu.make_async_copy(c1_hbm.at[pl.ds(base, UCH)], cb, sC).wait()
        pltpu.make_async_copy(p16_hbm.at[pl.ds(base, UCH)], pb, sP).wait()

        def rw(i, carry2):
            p = pb[i]
            for cc in range(NCLS // 16):
                sl = pl.ds(cc * 16, 16)
                uu[i, sl] = p * ab[i, sl] + cb[i, sl]
            return carry2

        lax.fori_loop(0, UCH, rw, 0)
        pltpu.sync_copy(uu, U_sp.at[pl.ds(base, UCH)])
        pltpu.sync_copy(zsrc, S_sp.at[pl.ds(base, UCH)])

    def update_phase():
        _uissue(0, 0)

        def upair(g, carry):
            _uissue(2 * g + 1, 1)
            _ucompute(2 * g, 0)

            @pl.when(g < NUCH // 2 - 1)
            def _():
                _uissue(2 * g + 2, 0)

            _ucompute(2 * g + 1, 1)
            return carry

        lax.fori_loop(0, NUCH // 2, upair, 0)

    def kiter(k, carry):
        scatter_phase()
        plsc.subcore_barrier()
        update_phase()
        plsc.subcore_barrier()
        return carry

    lax.fori_loop(0, K_ITERS - 1, kiter, 0)
    scatter_phase()
    plsc.subcore_barrier()

    pltpu.sync_copy(b_hbm, bbuf)
    for h in range(B_PER_TILE // 16):
        ob = s * B_PER_TILE + h * 16
        pltpu.sync_copy(idx_hbm.at[pl.ds(ob, 16)], idxbuf)
        pltpu.sync_copy(qy_hbm.at[idxbuf], qybuf)
        pltpu.sync_copy(S_sp.at[idxbuf], rdbuf)

        def rbody(i, carry):
            q = qybuf[i, pl.ds(NCLS, 16)]
            for cc in range(NCLS // 16):
                sl = pl.ds(cc * 16, 16)
                obuf[h * 16 + i, sl] = (q * rdbuf[i, sl] + qybuf[i, sl]
                                        + bbuf[sl])
            return carry

        lax.fori_loop(0, 16, rbody, 0)
    pltpu.sync_copy(obuf, out_hbm.at[pl.ds(s * B_PER_TILE, B_PER_TILE)])


_main_kernel = functools.partial(
    pl.kernel,
    out_type=jax.ShapeDtypeStruct((BATCH, NCLS), jnp.float32),
    mesh=_mesh,
    scratch_types=[
        pltpu.VMEM((IGRP * GRP, CHUNK), jnp.int32),
        pltpu.VMEM((IGRP * GRP, CHUNK), jnp.int32),
        pltpu.VMEM((CHUNK, NCLS), jnp.float32),
        pltpu.VMEM((CHUNK, NCLS), jnp.float32),
        pltpu.VMEM((CHUNK, NCLS), jnp.float32),
        pltpu.VMEM((CHUNK, NCLS), jnp.float32),
        pltpu.VMEM((UCH, NCLS), jnp.float32),
        pltpu.VMEM((UCH, NCLS), jnp.float32),
        pltpu.VMEM((UCH, NCLS), jnp.float32),
        pltpu.VMEM((UCH, NCLS), jnp.float32),
        pltpu.VMEM((UCH, NCLS), jnp.float32),
        pltpu.VMEM((UCH, NCLS), jnp.float32),
        pltpu.VMEM((UCH, 16), jnp.float32),
        pltpu.VMEM((UCH, 16), jnp.float32),
        pltpu.VMEM((16,), jnp.int32),
        pltpu.VMEM((16, 128), jnp.float32),
        pltpu.VMEM((16, NCLS), jnp.float32),
        pltpu.VMEM((B_PER_TILE, NCLS), jnp.float32),
        pltpu.VMEM((NCLS,), jnp.float32),
        pltpu.SemaphoreType.DMA,
        pltpu.SemaphoreType.DMA,
        pltpu.SemaphoreType.DMA,
        pltpu.SemaphoreType.DMA,
        pltpu.SemaphoreType.DMA,
        pltpu.SemaphoreType.DMA,
        pltpu.SemaphoreType.DMA,
        pltpu.SemaphoreType.DMA,
        pltpu.SemaphoreType.DMA,
        pltpu.SemaphoreType.DMA,
        pltpu.SemaphoreType.DMA,
        pltpu.SemaphoreType.DMA,
        pltpu.SemaphoreType.DMA,
        pltpu.SemaphoreType.DMA,
        pltpu.VMEM_SHARED((N_PAD, NCLS), jnp.float32),
        pltpu.VMEM_SHARED((N_PAD, NCLS), jnp.float32),
    ],
    compiler_params=_sc_params,
)(_main_body)


def kernel(X, idx, edge_index, emb, W, b):
    del X
    emb_pad = jnp.pad(emb, ((0, N_PAD - N_NODES), (0, 0)))
    row = edge_index[0].astype(jnp.int32)
    col = edge_index[1].astype(jnp.int32)
    padv = jnp.full((E_PAD - N_EDGES,), N_NODES, jnp.int32)
    row3 = jnp.concatenate([row, padv]).reshape(NSUB, EC_PER_TILE, CHUNK)
    col3 = jnp.concatenate([col, padv]).reshape(NSUB, EC_PER_TILE, CHUNK)
    idx32 = idx.astype(jnp.int32)

    dout, din = _deg_kernel(row3, col3)
    c1, p16, qy = _prep(emb_pad, W, dout.reshape(N_PAD, 1),
                        din.reshape(N_PAD, 1))
    out = _main_kernel(row3, col3, c1, p16, qy, idx32, b)
    return out


# speedup vs baseline: 2.0145x; 1.0894x over previous
import functools

import jax
import jax.numpy as jnp
from jax import lax
from jax.experimental import pallas as pl
from jax.experimental.pallas import tpu as pltpu
from jax.experimental.pallas import tpu_sc as plsc

N_NODES = 10000
N_EDGES = 320000
HIDDEN = 128
NCLS = 64
BATCH = 1024
ALPHA = 0.1
K_ITERS = 10

NCORE = 1
NSUB = 16
B_PER_TILE = BATCH // (NCORE * NSUB)
N_PAD = 10240
ROWS_PER_TILE = N_PAD // NSUB
CHUNK = 64
NBUF = 4
GLA = NBUF // 2
GRP = 32
IGRP = 2
NGRP = 10
EC_PER_TILE = GRP * NGRP
E_PAD = NSUB * EC_PER_TILE * CHUNK
UCH = 32
NUCH = ROWS_PER_TILE // UCH
TBLK = 1024

_mesh = plsc.VectorSubcoreMesh(
    core_axis_name="c", subcore_axis_name="s", num_cores=NCORE,
    num_subcores=NSUB)
_sc_params = pltpu.CompilerParams(use_tc_tiling_on_sc=False)


def _deg_body(row_hbm, col_hbm, dout_hbm, din_hbm, rbuf, cbuf, ones, zbuf,
              do_sp, di_sp):
    s = lax.axis_index("s")
    for i in range(CHUNK // 16):
        ones[pl.ds(i * 16, 16)] = jnp.full((16,), 1.0, jnp.float32)
    for i in range(ROWS_PER_TILE // 16):
        zbuf[pl.ds(i * 16, 16)] = jnp.zeros((16,), jnp.float32)
    sl = pl.ds(s * ROWS_PER_TILE, ROWS_PER_TILE)
    pltpu.sync_copy(zbuf, do_sp.at[sl])
    pltpu.sync_copy(zbuf, di_sp.at[sl])
    pltpu.sync_copy(row_hbm.at[s], rbuf)
    pltpu.sync_copy(col_hbm.at[s], cbuf)
    plsc.subcore_barrier()

    def body(j, carry):
        pltpu.sync_copy(ones, do_sp.at[rbuf.at[j]], add=True)
        pltpu.sync_copy(ones, di_sp.at[cbuf.at[j]], add=True)
        return carry

    lax.fori_loop(0, EC_PER_TILE, body, 0)
    plsc.subcore_barrier()
    pltpu.sync_copy(do_sp.at[sl], dout_hbm.at[sl])
    pltpu.sync_copy(di_sp.at[sl], din_hbm.at[sl])


_deg_kernel = functools.partial(
    pl.kernel,
    out_type=(jax.ShapeDtypeStruct((N_PAD,), jnp.float32),
              jax.ShapeDtypeStruct((N_PAD,), jnp.float32)),
    mesh=_mesh,
    scratch_types=[
        pltpu.VMEM((EC_PER_TILE, CHUNK), jnp.int32),
        pltpu.VMEM((EC_PER_TILE, CHUNK), jnp.int32),
        pltpu.VMEM((CHUNK,), jnp.float32),
        pltpu.VMEM((ROWS_PER_TILE,), jnp.float32),
        pltpu.VMEM_SHARED((N_PAD,), jnp.float32),
        pltpu.VMEM_SHARED((N_PAD,), jnp.float32),
    ],
    compiler_params=_sc_params,
)(_deg_body)


def _prep_body(emb_ref, w_ref, dout_ref, din_ref,
               c1_ref, p16_ref, qy_ref):
    x = emb_ref[...]
    ss = jnp.sum(x * x, axis=-1, keepdims=True)
    h = x / (jnp.sqrt(ss) + 1e-12)
    y0 = jnp.dot(h, w_ref[...], preferred_element_type=jnp.float32)
    din = din_ref[...]
    dout = dout_ref[...]
    rin = lax.rsqrt(jnp.where(din > 0, din, 1.0))
    rout = lax.rsqrt(jnp.where(dout > 0, dout, 1.0))
    c1_ref[...] = ALPHA * rin * y0
    p16_ref[...] = jnp.broadcast_to((1.0 - ALPHA) * rin * rout, (TBLK, 16))
    qy_ref[...] = jnp.concatenate([
        ALPHA * y0,
        jnp.broadcast_to((1.0 - ALPHA) * rout, (TBLK, 16)),
        jnp.zeros((TBLK, 48), jnp.float32),
    ], axis=1)


def _prep(emb_pad, w, dout, din):
    grid = (N_PAD // TBLK,)
    return pl.pallas_call(
        _prep_body,
        grid=grid,
        in_specs=[
            pl.BlockSpec((TBLK, HIDDEN), lambda i: (i, 0)),
            pl.BlockSpec((HIDDEN, NCLS), lambda i: (0, 0)),
            pl.BlockSpec((TBLK, 1), lambda i: (i, 0)),
            pl.BlockSpec((TBLK, 1), lambda i: (i, 0)),
        ],
        out_specs=[
            pl.BlockSpec((TBLK, NCLS), lambda i: (i, 0)),
            pl.BlockSpec((TBLK, 16), lambda i: (i, 0)),
            pl.BlockSpec((TBLK, 128), lambda i: (i, 0)),
        ],
        out_shape=[
            jax.ShapeDtypeStruct((N_PAD, NCLS), jnp.float32),
            jax.ShapeDtypeStruct((N_PAD, 16), jnp.float32),
            jax.ShapeDtypeStruct((N_PAD, 128), jnp.float32),
        ],
    )(emb_pad, w, dout, din)


def _main_body(row_hbm, col_hbm, c1_hbm, p16_hbm, qy_hbm,
               idx_hbm, b_hbm, out_hbm,
               rbufc, cbufc,
               gbuf0, gbuf1, gbuf2, gbuf3,
               abufA, abufB, ubufA, ubufB, c1bufA, c1bufB,
               p16bufA, p16bufB,
               idxbuf, qybuf, rdbuf, obuf, bbuf,
               sem0, sem1, sem2, sem3,
               ssem0, ssem1, ssem2, ssem3,
               usemSA, usemSB, usemCA, usemCB, usemPA, usemPB,
               U_sp, S_sp):
    s = lax.axis_index("s")
    rbase = s * ROWS_PER_TILE
    gb = (gbuf0, gbuf1, gbuf2, gbuf3)
    sm = (sem0, sem1, sem2, sem3)
    ssm = (ssem0, ssem1, ssem2, ssem3)
    ub = ((abufA, ubufA, c1bufA, p16bufA, usemSA, usemCA, usemPA),
          (abufB, ubufB, c1bufB, p16bufB, usemSB, usemCB, usemPB))

    def zb(i, carry):
        for cc in range(NCLS // 16):
            obuf[i, pl.ds(cc * 16, 16)] = jnp.zeros((16,), jnp.float32)
        return carry

    lax.fori_loop(0, B_PER_TILE, zb, 0)
    zsrc = obuf.at[pl.ds(0, UCH)]

    def init_chunk(t, carry):
        base = rbase + t * UCH
        pltpu.sync_copy(c1_hbm.at[pl.ds(base, UCH)], c1bufA)

        def rw(i, carry2):
            for cc in range(NCLS // 16):
                sl = pl.ds(cc * 16, 16)
                ubufA[i, sl] = c1bufA[i, sl] * (1.0 / ALPHA)
            return carry2

        lax.fori_loop(0, UCH, rw, 0)
        pltpu.sync_copy(ubufA, U_sp.at[pl.ds(base, UCH)])
        pltpu.sync_copy(zsrc, S_sp.at[pl.ds(base, UCH)])
        return carry

    lax.fori_loop(0, NUCH, init_chunk, 0)
    plsc.subcore_barrier()

    def scatter_phase():
        def sgroup(g, carry):
            pltpu.sync_copy(
                row_hbm.at[s, pl.ds(g * IGRP * GRP, IGRP * GRP)], rbufc)
            pltpu.sync_copy(
                col_hbm.at[s, pl.ds(g * IGRP * GRP, IGRP * GRP)], cbufc)
            for sub in range(IGRP):
                o = sub * GRP
                gd = [None] * NBUF
                sd = [None] * NBUF
                for j in range(GLA):
                    gd[j] = pltpu.async_copy(
                        U_sp.at[cbufc.at[o + j]], gb[j], sm[j])
                for j in range(GRP):
                    cur = j % NBUF
                    gd[cur].wait()
                    sd[cur] = pltpu.async_copy(
                        gb[cur], S_sp.at[rbufc.at[o + j]], ssm[cur],
                        add=True)
                    if j + GLA < GRP:
                        nx = (j + GLA) % NBUF
                        if sd[nx] is not None:
                            sd[nx].wait()
                        gd[nx] = pltpu.async_copy(
                            U_sp.at[cbufc.at[o + j + GLA]], gb[nx], sm[nx])
                for j in range(GRP - NBUF, GRP):
                    sd[j % NBUF].wait()
            return carry

        lax.fori_loop(0, NGRP // IGRP, sgroup, 0)

    def _uissue(t, bs):
        ab, _, cb, pb, sS, sC, sP = ub[bs]
        base = rbase + t * UCH
        pltpu.async_copy(S_sp.at[pl.ds(base, UCH)], ab, sS)
        pltpu.async_copy(c1_hbm.at[pl.ds(base, UCH)], cb, sC)
        pltpu.async_copy(p16_hbm.at[pl.ds(base, UCH)], pb, sP)

    def _ucompute(t, bs):
        ab, uu, cb, pb, sS, sC, sP = ub[bs]
        base = rbase + t * UCH
        pltpu.make_async_copy(S_sp.at[pl.ds(base, UCH)], ab, sS).wait()
        pltpu.make_async_copy(c1_hbm.at[pl.ds(base, UCH)], cb, sC).wait()
        pltpu.make_async_copy(p16_hbm.at[pl.ds(base, UCH)], pb, sP).wait()

        def rw(i, carry2):
            p = pb[i]
            for cc in range(NCLS // 16):
                sl = pl.ds(cc * 16, 16)
                uu[i, sl] = p * ab[i, sl] + cb[i, sl]
            return carry2

        lax.fori_loop(0, UCH, rw, 0)
        pltpu.sync_copy(uu, U_sp.at[pl.ds(base, UCH)])
        pltpu.sync_copy(zsrc, S_sp.at[pl.ds(base, UCH)])

    def update_phase():
        _uissue(0, 0)

        def upair(g, carry):
            _uissue(2 * g + 1, 1)
            _ucompute(2 * g, 0)

            @pl.when(g < NUCH // 2 - 1)
            def _():
                _uissue(2 * g + 2, 0)

            _ucompute(2 * g + 1, 1)
            return carry

        lax.fori_loop(0, NUCH // 2, upair, 0)

    def kiter(k, carry):
        scatter_phase()
        plsc.subcore_barrier()
        update_phase()
        plsc.subcore_barrier()
        return carry

    lax.fori_loop(0, K_ITERS - 1, kiter, 0)
    scatter_phase()
    plsc.subcore_barrier()

    pltpu.sync_copy(b_hbm, bbuf)
    for h in range(B_PER_TILE // 16):
        ob = s * B_PER_TILE + h * 16
        pltpu.sync_copy(idx_hbm.at[pl.ds(ob, 16)], idxbuf)
        pltpu.sync_copy(qy_hbm.at[idxbuf], qybuf)
        pltpu.sync_copy(S_sp.at[idxbuf], rdbuf)

        def rbody(i, carry):
            q = qybuf[i, pl.ds(NCLS, 16)]
            for cc in range(NCLS // 16):
                sl = pl.ds(cc * 16, 16)
                obuf[h * 16 + i, sl] = (q * rdbuf[i, sl] + qybuf[i, sl]
                                        + bbuf[sl])
            return carry

        lax.fori_loop(0, 16, rbody, 0)
    pltpu.sync_copy(obuf, out_hbm.at[pl.ds(s * B_PER_TILE, B_PER_TILE)])


_main_kernel = functools.partial(
    pl.kernel,
    out_type=jax.ShapeDtypeStruct((BATCH, NCLS), jnp.float32),
    mesh=_mesh,
    scratch_types=[
        pltpu.VMEM((IGRP * GRP, CHUNK), jnp.int32),
        pltpu.VMEM((IGRP * GRP, CHUNK), jnp.int32),
        pltpu.VMEM((CHUNK, NCLS), jnp.float32),
        pltpu.VMEM((CHUNK, NCLS), jnp.float32),
        pltpu.VMEM((CHUNK, NCLS), jnp.float32),
        pltpu.VMEM((CHUNK, NCLS), jnp.float32),
        pltpu.VMEM((UCH, NCLS), jnp.float32),
        pltpu.VMEM((UCH, NCLS), jnp.float32),
        pltpu.VMEM((UCH, NCLS), jnp.float32),
        pltpu.VMEM((UCH, NCLS), jnp.float32),
        pltpu.VMEM((UCH, NCLS), jnp.float32),
        pltpu.VMEM((UCH, NCLS), jnp.float32),
        pltpu.VMEM((UCH, 16), jnp.float32),
        pltpu.VMEM((UCH, 16), jnp.float32),
        pltpu.VMEM((16,), jnp.int32),
        pltpu.VMEM((16, 128), jnp.float32),
        pltpu.VMEM((16, NCLS), jnp.float32),
        pltpu.VMEM((B_PER_TILE, NCLS), jnp.float32),
        pltpu.VMEM((NCLS,), jnp.float32),
        pltpu.SemaphoreType.DMA,
        pltpu.SemaphoreType.DMA,
        pltpu.SemaphoreType.DMA,
        pltpu.SemaphoreType.DMA,
        pltpu.SemaphoreType.DMA,
        pltpu.SemaphoreType.DMA,
        pltpu.SemaphoreType.DMA,
        pltpu.SemaphoreType.DMA,
        pltpu.SemaphoreType.DMA,
        pltpu.SemaphoreType.DMA,
        pltpu.SemaphoreType.DMA,
        pltpu.SemaphoreType.DMA,
        pltpu.SemaphoreType.DMA,
        pltpu.SemaphoreType.DMA,
        pltpu.VMEM_SHARED((N_PAD, NCLS), jnp.float32),
        pltpu.VMEM_SHARED((N_PAD, NCLS), jnp.float32),
    ],
    compiler_params=_sc_params,
)(_main_body)


def kernel(X, idx, edge_index, emb, W, b):
    del X
    emb_pad = jnp.pad(emb, ((0, N_PAD - N_NODES), (0, 0)))
    row = edge_index[0].astype(jnp.int32)
    col = edge_index[1].astype(jnp.int32)
    padv = jnp.full((E_PAD - N_EDGES,), N_NODES, jnp.int32)
    row3 = jnp.concatenate([row, padv]).reshape(NSUB, EC_PER_TILE, CHUNK)
    col3 = jnp.concatenate([col, padv]).reshape(NSUB, EC_PER_TILE, CHUNK)
    idx32 = idx.astype(jnp.int32)

    dout, din = _deg_kernel(row3, col3)
    c1, p16, qy = _prep(emb_pad, W, dout.reshape(N_PAD, 1),
                        din.reshape(N_PAD, 1))
    out = _main_kernel(row3, col3, c1, p16, qy, idx32, b)
    return out


# speedup vs baseline: 2.1128x; 1.0488x over previous
import functools

import jax
import jax.numpy as jnp
from jax import lax
from jax.experimental import pallas as pl
from jax.experimental.pallas import tpu as pltpu
from jax.experimental.pallas import tpu_sc as plsc

N_NODES = 10000
N_EDGES = 320000
HIDDEN = 128
NCLS = 64
BATCH = 1024
ALPHA = 0.1
K_ITERS = 10

NCORE = 1
NSUB = 16
B_PER_TILE = BATCH // (NCORE * NSUB)
N_PAD = 10240
ROWS_PER_TILE = N_PAD // NSUB
CHUNK = 64
NBUF = 4
GLA = NBUF // 2
GRP = 64
IGRP = 1
NGRP = 5
EC_PER_TILE = GRP * NGRP
E_PAD = NSUB * EC_PER_TILE * CHUNK
UCH = 32
NUCH = ROWS_PER_TILE // UCH
TBLK = 1024

_mesh = plsc.VectorSubcoreMesh(
    core_axis_name="c", subcore_axis_name="s", num_cores=NCORE,
    num_subcores=NSUB)
_sc_params = pltpu.CompilerParams(use_tc_tiling_on_sc=False)


def _deg_body(row_hbm, col_hbm, dout_hbm, din_hbm, rbuf, cbuf, ones, zbuf,
              do_sp, di_sp):
    s = lax.axis_index("s")
    for i in range(CHUNK // 16):
        ones[pl.ds(i * 16, 16)] = jnp.full((16,), 1.0, jnp.float32)
    for i in range(ROWS_PER_TILE // 16):
        zbuf[pl.ds(i * 16, 16)] = jnp.zeros((16,), jnp.float32)
    sl = pl.ds(s * ROWS_PER_TILE, ROWS_PER_TILE)
    pltpu.sync_copy(zbuf, do_sp.at[sl])
    pltpu.sync_copy(zbuf, di_sp.at[sl])
    pltpu.sync_copy(row_hbm.at[s], rbuf)
    pltpu.sync_copy(col_hbm.at[s], cbuf)
    plsc.subcore_barrier()

    def body(j, carry):
        pltpu.sync_copy(ones, do_sp.at[rbuf.at[j]], add=True)
        pltpu.sync_copy(ones, di_sp.at[cbuf.at[j]], add=True)
        return carry

    lax.fori_loop(0, EC_PER_TILE, body, 0)
    plsc.subcore_barrier()
    pltpu.sync_copy(do_sp.at[sl], dout_hbm.at[sl])
    pltpu.sync_copy(di_sp.at[sl], din_hbm.at[sl])


_deg_kernel = functools.partial(
    pl.kernel,
    out_type=(jax.ShapeDtypeStruct((N_PAD,), jnp.float32),
              jax.ShapeDtypeStruct((N_PAD,), jnp.float32)),
    mesh=_mesh,
    scratch_types=[
        pltpu.VMEM((EC_PER_TILE, CHUNK), jnp.int32),
        pltpu.VMEM((EC_PER_TILE, CHUNK), jnp.int32),
        pltpu.VMEM((CHUNK,), jnp.float32),
        pltpu.VMEM((ROWS_PER_TILE,), jnp.float32),
        pltpu.VMEM_SHARED((N_PAD,), jnp.float32),
        pltpu.VMEM_SHARED((N_PAD,), jnp.float32),
    ],
    compiler_params=_sc_params,
)(_deg_body)


def _prep_body(emb_ref, w_ref, dout_ref, din_ref,
               c1_ref, p16_ref, qy_ref):
    x = emb_ref[...]
    ss = jnp.sum(x * x, axis=-1, keepdims=True)
    h = x / (jnp.sqrt(ss) + 1e-12)
    y0 = jnp.dot(h, w_ref[...], preferred_element_type=jnp.float32)
    din = din_ref[...]
    dout = dout_ref[...]
    rin = lax.rsqrt(jnp.where(din > 0, din, 1.0))
    rout = lax.rsqrt(jnp.where(dout > 0, dout, 1.0))
    c1_ref[...] = ALPHA * rin * y0
    p16_ref[...] = jnp.broadcast_to((1.0 - ALPHA) * rin * rout, (TBLK, 16))
    qy_ref[...] = jnp.concatenate([
        ALPHA * y0,
        jnp.broadcast_to((1.0 - ALPHA) * rout, (TBLK, 16)),
        jnp.zeros((TBLK, 48), jnp.float32),
    ], axis=1)


def _prep(emb_pad, w, dout, din):
    grid = (N_PAD // TBLK,)
    return pl.pallas_call(
        _prep_body,
        grid=grid,
        in_specs=[
            pl.BlockSpec((TBLK, HIDDEN), lambda i: (i, 0)),
            pl.BlockSpec((HIDDEN, NCLS), lambda i: (0, 0)),
            pl.BlockSpec((TBLK, 1), lambda i: (i, 0)),
            pl.BlockSpec((TBLK, 1), lambda i: (i, 0)),
        ],
        out_specs=[
            pl.BlockSpec((TBLK, NCLS), lambda i: (i, 0)),
            pl.BlockSpec((TBLK, 16), lambda i: (i, 0)),
            pl.BlockSpec((TBLK, 128), lambda i: (i, 0)),
        ],
        out_shape=[
            jax.ShapeDtypeStruct((N_PAD, NCLS), jnp.float32),
            jax.ShapeDtypeStruct((N_PAD, 16), jnp.float32),
            jax.ShapeDtypeStruct((N_PAD, 128), jnp.float32),
        ],
    )(emb_pad, w, dout, din)


def _main_body(row_hbm, col_hbm, c1_hbm, p16_hbm, qy_hbm,
               idx_hbm, b_hbm, out_hbm,
               rbufc, cbufc,
               gbuf0, gbuf1, gbuf2, gbuf3,
               abufA, abufB, ubufA, ubufB, c1bufA, c1bufB,
               p16bufA, p16bufB,
               idxbuf, qybuf, rdbuf, obuf, bbuf,
               sem0, sem1, sem2, sem3,
               ssem0, ssem1, ssem2, ssem3,
               usemSA, usemSB, usemCA, usemCB, usemPA, usemPB,
               U_sp, S_sp):
    s = lax.axis_index("s")
    rbase = s * ROWS_PER_TILE
    gb = (gbuf0, gbuf1, gbuf2, gbuf3)
    sm = (sem0, sem1, sem2, sem3)
    ssm = (ssem0, ssem1, ssem2, ssem3)
    ub = ((abufA, ubufA, c1bufA, p16bufA, usemSA, usemCA, usemPA),
          (abufB, ubufB, c1bufB, p16bufB, usemSB, usemCB, usemPB))

    def zb(i, carry):
        for cc in range(NCLS // 16):
            obuf[i, pl.ds(cc * 16, 16)] = jnp.zeros((16,), jnp.float32)
        return carry

    lax.fori_loop(0, B_PER_TILE, zb, 0)
    zsrc = obuf.at[pl.ds(0, UCH)]

    def init_chunk(t, carry):
        base = rbase + t * UCH
        pltpu.sync_copy(c1_hbm.at[pl.ds(base, UCH)], c1bufA)

        def rw(i, carry2):
            for cc in range(NCLS // 16):
                sl = pl.ds(cc * 16, 16)
                ubufA[i, sl] = c1bufA[i, sl] * (1.0 / ALPHA)
            return carry2

        lax.fori_loop(0, UCH, rw, 0)
        pltpu.sync_copy(ubufA, U_sp.at[pl.ds(base, UCH)])
        pltpu.sync_copy(zsrc, S_sp.at[pl.ds(base, UCH)])
        return carry

    lax.fori_loop(0, NUCH, init_chunk, 0)
    plsc.subcore_barrier()

    def scatter_phase():
        def sgroup(g, carry):
            pltpu.sync_copy(
                row_hbm.at[s, pl.ds(g * IGRP * GRP, IGRP * GRP)], rbufc)
            pltpu.sync_copy(
                col_hbm.at[s, pl.ds(g * IGRP * GRP, IGRP * GRP)], cbufc)
            for sub in range(IGRP):
                o = sub * GRP
                gd = [None] * NBUF
                sd = [None] * NBUF
                for j in range(GLA):
                    gd[j] = pltpu.async_copy(
                        U_sp.at[cbufc.at[o + j]], gb[j], sm[j])
                for j in range(GRP):
                    cur = j % NBUF
                    gd[cur].wait()
                    sd[cur] = pltpu.async_copy(
                        gb[cur], S_sp.at[rbufc.at[o + j]], ssm[cur],
                        add=True)
                    if j + GLA < GRP:
                        nx = (j + GLA) % NBUF
                        if sd[nx] is not None:
                            sd[nx].wait()
                        gd[nx] = pltpu.async_copy(
                            U_sp.at[cbufc.at[o + j + GLA]], gb[nx], sm[nx])
                for j in range(GRP - NBUF, GRP):
                    sd[j % NBUF].wait()
            return carry

        lax.fori_loop(0, NGRP // IGRP, sgroup, 0)

    def _uissue(t, bs):
        ab, _, cb, pb, sS, sC, sP = ub[bs]
        base = rbase + t * UCH
        pltpu.async_copy(S_sp.at[pl.ds(base, UCH)], ab, sS)
        pltpu.async_copy(c1_hbm.at[pl.ds(base, UCH)], cb, sC)
        pltpu.async_copy(p16_hbm.at[pl.ds(base, UCH)], pb, sP)

    def _ucompute(t, bs):
        ab, uu, cb, pb, sS, sC, sP = ub[bs]
        base = rbase + t * UCH
        pltpu.make_async_copy(S_sp.at[pl.ds(base, UCH)], ab, sS).wait()
        pltpu.make_async_copy(c1_hbm.at[pl.ds(base, UCH)], cb, sC).wait()
        pltpu.make_async_copy(p16_hbm.at[pl.ds(base, UCH)], pb, sP).wait()

        def rw(i, carry2):
            p = pb[i]
            for cc in range(NCLS // 16):
                sl = pl.ds(cc * 16, 16)
                uu[i, sl] = p * ab[i, sl] + cb[i, sl]
            return carry2

        lax.fori_loop(0, UCH, rw, 0)
        pltpu.sync_copy(uu, U_sp.at[pl.ds(base, UCH)])
        pltpu.sync_copy(zsrc, S_sp.at[pl.ds(base, UCH)])

    def update_phase():
        _uissue(0, 0)

        def upair(g, carry):
            _uissue(2 * g + 1, 1)
            _ucompute(2 * g, 0)

            @pl.when(g < NUCH // 2 - 1)
            def _():
                _uissue(2 * g + 2, 0)

            _ucompute(2 * g + 1, 1)
            return carry

        lax.fori_loop(0, NUCH // 2, upair, 0)

    def kiter(k, carry):
        scatter_phase()
        plsc.subcore_barrier()
        update_phase()
        plsc.subcore_barrier()
        return carry

    lax.fori_loop(0, K_ITERS - 1, kiter, 0)
    scatter_phase()
    plsc.subcore_barrier()

    pltpu.sync_copy(b_hbm, bbuf)
    for h in range(B_PER_TILE // 16):
        ob = s * B_PER_TILE + h * 16
        pltpu.sync_copy(idx_hbm.at[pl.ds(ob, 16)], idxbuf)
        pltpu.sync_copy(qy_hbm.at[idxbuf], qybuf)
        pltpu.sync_copy(S_sp.at[idxbuf], rdbuf)

        def rbody(i, carry):
            q = qybuf[i, pl.ds(NCLS, 16)]
            for cc in range(NCLS // 16):
                sl = pl.ds(cc * 16, 16)
                obuf[h * 16 + i, sl] = (q * rdbuf[i, sl] + qybuf[i, sl]
                                        + bbuf[sl])
            return carry

        lax.fori_loop(0, 16, rbody, 0)
    pltpu.sync_copy(obuf, out_hbm.at[pl.ds(s * B_PER_TILE, B_PER_TILE)])


_main_kernel = functools.partial(
    pl.kernel,
    out_type=jax.ShapeDtypeStruct((BATCH, NCLS), jnp.float32),
    mesh=_mesh,
    scratch_types=[
        pltpu.VMEM((IGRP * GRP, CHUNK), jnp.int32),
        pltpu.VMEM((IGRP * GRP, CHUNK), jnp.int32),
        pltpu.VMEM((CHUNK, NCLS), jnp.float32),
        pltpu.VMEM((CHUNK, NCLS), jnp.float32),
        pltpu.VMEM((CHUNK, NCLS), jnp.float32),
        pltpu.VMEM((CHUNK, NCLS), jnp.float32),
        pltpu.VMEM((UCH, NCLS), jnp.float32),
        pltpu.VMEM((UCH, NCLS), jnp.float32),
        pltpu.VMEM((UCH, NCLS), jnp.float32),
        pltpu.VMEM((UCH, NCLS), jnp.float32),
        pltpu.VMEM((UCH, NCLS), jnp.float32),
        pltpu.VMEM((UCH, NCLS), jnp.float32),
        pltpu.VMEM((UCH, 16), jnp.float32),
        pltpu.VMEM((UCH, 16), jnp.float32),
        pltpu.VMEM((16,), jnp.int32),
        pltpu.VMEM((16, 128), jnp.float32),
        pltpu.VMEM((16, NCLS), jnp.float32),
        pltpu.VMEM((B_PER_TILE, NCLS), jnp.float32),
        pltpu.VMEM((NCLS,), jnp.float32),
        pltpu.SemaphoreType.DMA,
        pltpu.SemaphoreType.DMA,
        pltpu.SemaphoreType.DMA,
        pltpu.SemaphoreType.DMA,
        pltpu.SemaphoreType.DMA,
        pltpu.SemaphoreType.DMA,
        pltpu.SemaphoreType.DMA,
        pltpu.SemaphoreType.DMA,
        pltpu.SemaphoreType.DMA,
        pltpu.SemaphoreType.DMA,
        pltpu.SemaphoreType.DMA,
        pltpu.SemaphoreType.DMA,
        pltpu.SemaphoreType.DMA,
        pltpu.SemaphoreType.DMA,
        pltpu.VMEM_SHARED((N_PAD, NCLS), jnp.float32),
        pltpu.VMEM_SHARED((N_PAD, NCLS), jnp.float32),
    ],
    compiler_params=_sc_params,
)(_main_body)


def kernel(X, idx, edge_index, emb, W, b):
    del X
    emb_pad = jnp.pad(emb, ((0, N_PAD - N_NODES), (0, 0)))
    row = edge_index[0].astype(jnp.int32)
    col = edge_index[1].astype(jnp.int32)
    padv = jnp.full((E_PAD - N_EDGES,), N_NODES, jnp.int32)
    row3 = jnp.concatenate([row, padv]).reshape(NSUB, EC_PER_TILE, CHUNK)
    col3 = jnp.concatenate([col, padv]).reshape(NSUB, EC_PER_TILE, CHUNK)
    idx32 = idx.astype(jnp.int32)

    dout, din = _deg_kernel(row3, col3)
    c1, p16, qy = _prep(emb_pad, W, dout.reshape(N_PAD, 1),
                        din.reshape(N_PAD, 1))
    out = _main_kernel(row3, col3, c1, p16, qy, idx32, b)
    return out


# speedup vs baseline: 2.2884x; 1.0831x over previous
import functools

import jax
import jax.numpy as jnp
from jax import lax
from jax.experimental import pallas as pl
from jax.experimental.pallas import tpu as pltpu
from jax.experimental.pallas import tpu_sc as plsc

N_NODES = 10000
N_EDGES = 320000
HIDDEN = 128
NCLS = 64
BATCH = 1024
ALPHA = 0.1
K_ITERS = 10

NCORE = 1
NSUB = 16
B_PER_TILE = BATCH // (NCORE * NSUB)
N_PAD = 10240
ROWS_PER_TILE = N_PAD // NSUB
CHUNK = 64
NBUF = 4
GLA = NBUF // 2
GRP = 32
NPAIR = 5
NGRP = 2 * NPAIR
EC_PER_TILE = GRP * NGRP
E_PAD = NSUB * EC_PER_TILE * CHUNK
UCH = 32
NUCH = ROWS_PER_TILE // UCH
TBLK = 1024

_mesh = plsc.VectorSubcoreMesh(
    core_axis_name="c", subcore_axis_name="s", num_cores=NCORE,
    num_subcores=NSUB)
_sc_params = pltpu.CompilerParams(use_tc_tiling_on_sc=False)


def _deg_body(row_hbm, col_hbm, dout_hbm, din_hbm, rbuf, cbuf, ones, zbuf,
              dsem0, dsem1, do_sp, di_sp):
    s = lax.axis_index("s")
    for i in range(CHUNK // 16):
        ones[pl.ds(i * 16, 16)] = jnp.full((16,), 1.0, jnp.float32)
    for i in range(ROWS_PER_TILE // 16):
        zbuf[pl.ds(i * 16, 16)] = jnp.zeros((16,), jnp.float32)
    sl = pl.ds(s * ROWS_PER_TILE, ROWS_PER_TILE)
    pltpu.sync_copy(zbuf, do_sp.at[sl])
    pltpu.sync_copy(zbuf, di_sp.at[sl])
    pltpu.sync_copy(row_hbm.at[s], rbuf)
    pltpu.sync_copy(col_hbm.at[s], cbuf)
    plsc.subcore_barrier()

    def body(j, carry):
        d0 = pltpu.async_copy(ones, do_sp.at[rbuf.at[j]], dsem0, add=True)
        d1 = pltpu.async_copy(ones, di_sp.at[cbuf.at[j]], dsem1, add=True)
        d0.wait()
        d1.wait()
        return carry

    lax.fori_loop(0, EC_PER_TILE, body, 0)
    plsc.subcore_barrier()
    pltpu.sync_copy(do_sp.at[sl], dout_hbm.at[sl])
    pltpu.sync_copy(di_sp.at[sl], din_hbm.at[sl])


_deg_kernel = functools.partial(
    pl.kernel,
    out_type=(jax.ShapeDtypeStruct((N_PAD,), jnp.float32),
              jax.ShapeDtypeStruct((N_PAD,), jnp.float32)),
    mesh=_mesh,
    scratch_types=[
        pltpu.VMEM((EC_PER_TILE, CHUNK), jnp.int32),
        pltpu.VMEM((EC_PER_TILE, CHUNK), jnp.int32),
        pltpu.VMEM((CHUNK,), jnp.float32),
        pltpu.VMEM((ROWS_PER_TILE,), jnp.float32),
        pltpu.SemaphoreType.DMA,
        pltpu.SemaphoreType.DMA,
        pltpu.VMEM_SHARED((N_PAD,), jnp.float32),
        pltpu.VMEM_SHARED((N_PAD,), jnp.float32),
    ],
    compiler_params=_sc_params,
)(_deg_body)


def _prep_body(emb_ref, w_ref, dout_ref, din_ref,
               c1_ref, p16_ref, qy_ref):
    x = emb_ref[...]
    ss = jnp.sum(x * x, axis=-1, keepdims=True)
    h = x / (jnp.sqrt(ss) + 1e-12)
    y0 = jnp.dot(h, w_ref[...], preferred_element_type=jnp.float32)
    din = din_ref[...]
    dout = dout_ref[...]
    rin = lax.rsqrt(jnp.where(din > 0, din, 1.0))
    rout = lax.rsqrt(jnp.where(dout > 0, dout, 1.0))
    c1_ref[...] = ALPHA * rin * y0
    p16_ref[...] = jnp.broadcast_to((1.0 - ALPHA) * rin * rout, (TBLK, 16))
    qy_ref[...] = jnp.concatenate([
        ALPHA * y0,
        jnp.broadcast_to((1.0 - ALPHA) * rout, (TBLK, 16)),
        jnp.zeros((TBLK, 48), jnp.float32),
    ], axis=1)


def _prep(emb_pad, w, dout, din):
    grid = (N_PAD // TBLK,)
    return pl.pallas_call(
        _prep_body,
        grid=grid,
        in_specs=[
            pl.BlockSpec((TBLK, HIDDEN), lambda i: (i, 0)),
            pl.BlockSpec((HIDDEN, NCLS), lambda i: (0, 0)),
            pl.BlockSpec((TBLK, 1), lambda i: (i, 0)),
            pl.BlockSpec((TBLK, 1), lambda i: (i, 0)),
        ],
        out_specs=[
            pl.BlockSpec((TBLK, NCLS), lambda i: (i, 0)),
            pl.BlockSpec((TBLK, 16), lambda i: (i, 0)),
            pl.BlockSpec((TBLK, 128), lambda i: (i, 0)),
        ],
        out_shape=[
            jax.ShapeDtypeStruct((N_PAD, NCLS), jnp.float32),
            jax.ShapeDtypeStruct((N_PAD, 16), jnp.float32),
            jax.ShapeDtypeStruct((N_PAD, 128), jnp.float32),
        ],
    )(emb_pad, w, dout, din)


def _main_body(row_hbm, col_hbm, c1_hbm, p16_hbm, qy_hbm,
               idx_hbm, b_hbm, out_hbm,
               rbufc, cbufc, rbufcB, cbufcB,
               gbuf0, gbuf1, gbuf2, gbuf3,
               abufA, abufB, ubufA, ubufB, c1bufA, c1bufB,
               p16bufA, p16bufB,
               idxbuf, qybuf, rdbuf, obuf, bbuf,
               sem0, sem1, sem2, sem3,
               ssem0, ssem1, ssem2, ssem3,
               usemSA, usemSB, usemCA, usemCB, usemPA, usemPB,
               isem0, isem1,
               U_sp, S_sp):
    s = lax.axis_index("s")
    rbase = s * ROWS_PER_TILE
    gb = (gbuf0, gbuf1, gbuf2, gbuf3)
    sm = (sem0, sem1, sem2, sem3)
    ssm = (ssem0, ssem1, ssem2, ssem3)
    ub = ((abufA, ubufA, c1bufA, p16bufA, usemSA, usemCA, usemPA),
          (abufB, ubufB, c1bufB, p16bufB, usemSB, usemCB, usemPB))

    def zb(i, carry):
        for cc in range(NCLS // 16):
            obuf[i, pl.ds(cc * 16, 16)] = jnp.zeros((16,), jnp.float32)
        return carry

    lax.fori_loop(0, B_PER_TILE, zb, 0)
    zsrc = obuf.at[pl.ds(0, UCH)]

    def init_chunk(t, carry):
        base = rbase + t * UCH
        pltpu.sync_copy(c1_hbm.at[pl.ds(base, UCH)], c1bufA)

        def rw(i, carry2):
            for cc in range(NCLS // 16):
                sl = pl.ds(cc * 16, 16)
                ubufA[i, sl] = c1bufA[i, sl] * (1.0 / ALPHA)
            return carry2

        lax.fori_loop(0, UCH, rw, 0)
        pltpu.sync_copy(ubufA, U_sp.at[pl.ds(base, UCH)])
        pltpu.sync_copy(zsrc, S_sp.at[pl.ds(base, UCH)])
        return carry

    lax.fori_loop(0, NUCH, init_chunk, 0)
    plsc.subcore_barrier()

    def scatter_phase():
        PAIR = 2 * GRP

        def irow(which, jj):
            rb, cb2 = (rbufc, cbufc) if jj < GRP else (rbufcB, cbufcB)
            return (rb.at[jj % GRP], cb2.at[jj % GRP])

        def g_issue(jj, blk_r, blk_c):
            del blk_r
            return pltpu.async_copy(
                U_sp.at[blk_c], gb[jj % NBUF], sm[jj % NBUF])

        pltpu.sync_copy(row_hbm.at[s, pl.ds(0, GRP)], rbufc)
        pltpu.sync_copy(col_hbm.at[s, pl.ds(0, GRP)], cbufc)
        for j in range(GLA):
            _, bc = irow(0, j)
            g_issue(j, None, bc)

        def pair_body(g, carry):
            base = g * PAIR
            pltpu.async_copy(
                row_hbm.at[s, pl.ds(base + GRP, GRP)], rbufcB, isem0)
            pltpu.async_copy(
                col_hbm.at[s, pl.ds(base + GRP, GRP)], cbufcB, isem1)
            for j in range(PAIR):
                cur = j % NBUF
                br, bc = irow(0, j)
                pltpu.make_async_copy(U_sp.at[bc], gb[cur], sm[cur]).wait()
                pltpu.async_copy(
                    gb[cur], S_sp.at[br], ssm[cur], add=True)
                if j == GRP - GLA:
                    pltpu.make_async_copy(
                        row_hbm.at[s, pl.ds(base + GRP, GRP)], rbufcB,
                        isem0).wait()
                    pltpu.make_async_copy(
                        col_hbm.at[s, pl.ds(base + GRP, GRP)], cbufcB,
                        isem1).wait()
                if j == GRP + GLA:
                    @pl.when(g < NPAIR - 1)
                    def _():
                        pltpu.async_copy(
                            row_hbm.at[s, pl.ds(base + PAIR, GRP)],
                            rbufc, isem0)
                        pltpu.async_copy(
                            col_hbm.at[s, pl.ds(base + PAIR, GRP)],
                            cbufc, isem1)
                nxt = j + GLA
                if nxt < PAIR:
                    nb = nxt % NBUF
                    if j < GLA:
                        @pl.when(g > 0)
                        def _():
                            pltpu.make_async_copy(
                                gb[nb], S_sp.at[br], ssm[nb]).wait()
                    else:
                        pltpu.make_async_copy(
                            gb[nb], S_sp.at[br], ssm[nb]).wait()
                    _, bc2 = irow(0, nxt)
                    g_issue(nxt, None, bc2)
                else:
                    @pl.when(g < NPAIR - 1)
                    def _():
                        if nxt == PAIR:
                            pltpu.make_async_copy(
                                row_hbm.at[s, pl.ds(base + PAIR, GRP)],
                                rbufc, isem0).wait()
                            pltpu.make_async_copy(
                                col_hbm.at[s, pl.ds(base + PAIR, GRP)],
                                cbufc, isem1).wait()
                        nb = nxt % NBUF
                        pltpu.make_async_copy(
                            gb[nb], S_sp.at[br], ssm[nb]).wait()
                        g_issue(nxt, None, cbufc.at[nxt % GRP])
            return carry

        lax.fori_loop(0, NPAIR, pair_body, 0)
        for k in range(NBUF):
            pltpu.make_async_copy(
                gb[k], S_sp.at[rbufc.at[k]], ssm[k]).wait()

    def _uissue(t, bs):
        ab, _, cb, pb, sS, sC, sP = ub[bs]
        base = rbase + t * UCH
        pltpu.async_copy(S_sp.at[pl.ds(base, UCH)], ab, sS)
        pltpu.async_copy(c1_hbm.at[pl.ds(base, UCH)], cb, sC)
        pltpu.async_copy(p16_hbm.at[pl.ds(base, UCH)], pb, sP)

    def _ucompute(t, bs):
        ab, uu, cb, pb, sS, sC, sP = ub[bs]
        base = rbase + t * UCH
        pltpu.make_async_copy(S_sp.at[pl.ds(base, UCH)], ab, sS).wait()
        pltpu.make_async_copy(c1_hbm.at[pl.ds(base, UCH)], cb, sC).wait()
        pltpu.make_async_copy(p16_hbm.at[pl.ds(base, UCH)], pb, sP).wait()

        def rw(i, carry2):
            p = pb[i]
            for cc in range(NCLS // 16):
                sl = pl.ds(cc * 16, 16)
                uu[i, sl] = p * ab[i, sl] + cb[i, sl]
            return carry2

        lax.fori_loop(0, UCH, rw, 0)
        pltpu.sync_copy(uu, U_sp.at[pl.ds(base, UCH)])
        pltpu.sync_copy(zsrc, S_sp.at[pl.ds(base, UCH)])

    def update_phase():
        _uissue(0, 0)

        def upair(g, carry):
            _uissue(2 * g + 1, 1)
            _ucompute(2 * g, 0)

            @pl.when(g < NUCH // 2 - 1)
            def _():
                _uissue(2 * g + 2, 0)

            _ucompute(2 * g + 1, 1)
            return carry

        lax.fori_loop(0, NUCH // 2, upair, 0)

    def kiter(k, carry):
        scatter_phase()
        plsc.subcore_barrier()
        update_phase()
        plsc.subcore_barrier()
        return carry

    lax.fori_loop(0, K_ITERS - 1, kiter, 0)
    scatter_phase()
    plsc.subcore_barrier()

    pltpu.sync_copy(b_hbm, bbuf)
    for h in range(B_PER_TILE // 16):
        ob = s * B_PER_TILE + h * 16
        pltpu.sync_copy(idx_hbm.at[pl.ds(ob, 16)], idxbuf)
        pltpu.sync_copy(qy_hbm.at[idxbuf], qybuf)
        pltpu.sync_copy(S_sp.at[idxbuf], rdbuf)

        def rbody(i, carry):
            q = qybuf[i, pl.ds(NCLS, 16)]
            for cc in range(NCLS // 16):
                sl = pl.ds(cc * 16, 16)
                obuf[h * 16 + i, sl] = (q * rdbuf[i, sl] + qybuf[i, sl]
                                        + bbuf[sl])
            return carry

        lax.fori_loop(0, 16, rbody, 0)
    pltpu.sync_copy(obuf, out_hbm.at[pl.ds(s * B_PER_TILE, B_PER_TILE)])


_main_kernel = functools.partial(
    pl.kernel,
    out_type=jax.ShapeDtypeStruct((BATCH, NCLS), jnp.float32),
    mesh=_mesh,
    scratch_types=[
        pltpu.VMEM((GRP, CHUNK), jnp.int32),
        pltpu.VMEM((GRP, CHUNK), jnp.int32),
        pltpu.VMEM((GRP, CHUNK), jnp.int32),
        pltpu.VMEM((GRP, CHUNK), jnp.int32),
        pltpu.VMEM((CHUNK, NCLS), jnp.float32),
        pltpu.VMEM((CHUNK, NCLS), jnp.float32),
        pltpu.VMEM((CHUNK, NCLS), jnp.float32),
        pltpu.VMEM((CHUNK, NCLS), jnp.float32),
        pltpu.VMEM((UCH, NCLS), jnp.float32),
        pltpu.VMEM((UCH, NCLS), jnp.float32),
        pltpu.VMEM((UCH, NCLS), jnp.float32),
        pltpu.VMEM((UCH, NCLS), jnp.float32),
        pltpu.VMEM((UCH, NCLS), jnp.float32),
        pltpu.VMEM((UCH, NCLS), jnp.float32),
        pltpu.VMEM((UCH, 16), jnp.float32),
        pltpu.VMEM((UCH, 16), jnp.float32),
        pltpu.VMEM((16,), jnp.int32),
        pltpu.VMEM((16, 128), jnp.float32),
        pltpu.VMEM((16, NCLS), jnp.float32),
        pltpu.VMEM((B_PER_TILE, NCLS), jnp.float32),
        pltpu.VMEM((NCLS,), jnp.float32),
        pltpu.SemaphoreType.DMA,
        pltpu.SemaphoreType.DMA,
        pltpu.SemaphoreType.DMA,
        pltpu.SemaphoreType.DMA,
        pltpu.SemaphoreType.DMA,
        pltpu.SemaphoreType.DMA,
        pltpu.SemaphoreType.DMA,
        pltpu.SemaphoreType.DMA,
        pltpu.SemaphoreType.DMA,
        pltpu.SemaphoreType.DMA,
        pltpu.SemaphoreType.DMA,
        pltpu.SemaphoreType.DMA,
        pltpu.SemaphoreType.DMA,
        pltpu.SemaphoreType.DMA,
        pltpu.SemaphoreType.DMA,
        pltpu.SemaphoreType.DMA,
        pltpu.VMEM_SHARED((N_PAD, NCLS), jnp.float32),
        pltpu.VMEM_SHARED((N_PAD, NCLS), jnp.float32),
    ],
    compiler_params=_sc_params,
)(_main_body)


def kernel(X, idx, edge_index, emb, W, b):
    del X
    emb_pad = jnp.pad(emb, ((0, N_PAD - N_NODES), (0, 0)))
    row = edge_index[0].astype(jnp.int32)
    col = edge_index[1].astype(jnp.int32)
    padv = jnp.full((E_PAD - N_EDGES,), N_NODES, jnp.int32)
    row3 = jnp.concatenate([row, padv]).reshape(NSUB, EC_PER_TILE, CHUNK)
    col3 = jnp.concatenate([col, padv]).reshape(NSUB, EC_PER_TILE, CHUNK)
    idx32 = idx.astype(jnp.int32)

    dout, din = _deg_kernel(row3, col3)
    c1, p16, qy = _prep(emb_pad, W, dout.reshape(N_PAD, 1),
                        din.reshape(N_PAD, 1))
    out = _main_kernel(row3, col3, c1, p16, qy, idx32, b)
    return out


# speedup vs baseline: 2.3020x; 1.0059x over previous
import functools

import jax
import jax.numpy as jnp
from jax import lax
from jax.experimental import pallas as pl
from jax.experimental.pallas import tpu as pltpu
from jax.experimental.pallas import tpu_sc as plsc

N_NODES = 10000
N_EDGES = 320000
HIDDEN = 128
NCLS = 64
BATCH = 1024
ALPHA = 0.1
K_ITERS = 10

NCORE = 1
NSUB = 16
B_PER_TILE = BATCH // (NCORE * NSUB)
N_PAD = 10240
ROWS_PER_TILE = N_PAD // NSUB
CHUNK = 64
NBUF = 4
GLA = NBUF // 2
GRP = 32
NPAIR = 5
NGRP = 2 * NPAIR
EC_PER_TILE = GRP * NGRP
E_PAD = NSUB * EC_PER_TILE * CHUNK
UCH = 32
NUCH = ROWS_PER_TILE // UCH
TBLK = 1024

_mesh = plsc.VectorSubcoreMesh(
    core_axis_name="c", subcore_axis_name="s", num_cores=NCORE,
    num_subcores=NSUB)
_sc_params = pltpu.CompilerParams(use_tc_tiling_on_sc=False)


def _deg_body(row_hbm, col_hbm, dout_hbm, din_hbm, rbuf, cbuf, ones, zbuf,
              dsem0, dsem1, dsem2, dsem3, dsem4, dsem5, dsem6, dsem7,
              do_sp, di_sp):
    s = lax.axis_index("s")
    dsm = (dsem0, dsem1, dsem2, dsem3, dsem4, dsem5, dsem6, dsem7)
    for i in range(CHUNK // 16):
        ones[pl.ds(i * 16, 16)] = jnp.full((16,), 1.0, jnp.float32)
    for i in range(ROWS_PER_TILE // 16):
        zbuf[pl.ds(i * 16, 16)] = jnp.zeros((16,), jnp.float32)
    sl = pl.ds(s * ROWS_PER_TILE, ROWS_PER_TILE)
    pltpu.sync_copy(zbuf, do_sp.at[sl])
    pltpu.sync_copy(zbuf, di_sp.at[sl])
    pltpu.sync_copy(row_hbm.at[s], rbuf)
    pltpu.sync_copy(col_hbm.at[s], cbuf)
    plsc.subcore_barrier()

    def body(jj, carry):
        ds = []
        for k in range(4):
            j = jj * 4 + k
            ds.append(pltpu.async_copy(
                ones, do_sp.at[rbuf.at[j]], dsm[2 * k], add=True))
            ds.append(pltpu.async_copy(
                ones, di_sp.at[cbuf.at[j]], dsm[2 * k + 1], add=True))
        for d in ds:
            d.wait()
        return carry

    lax.fori_loop(0, EC_PER_TILE // 4, body, 0)
    plsc.subcore_barrier()
    pltpu.sync_copy(do_sp.at[sl], dout_hbm.at[sl])
    pltpu.sync_copy(di_sp.at[sl], din_hbm.at[sl])


_deg_kernel = functools.partial(
    pl.kernel,
    out_type=(jax.ShapeDtypeStruct((N_PAD,), jnp.float32),
              jax.ShapeDtypeStruct((N_PAD,), jnp.float32)),
    mesh=_mesh,
    scratch_types=[
        pltpu.VMEM((EC_PER_TILE, CHUNK), jnp.int32),
        pltpu.VMEM((EC_PER_TILE, CHUNK), jnp.int32),
        pltpu.VMEM((CHUNK,), jnp.float32),
        pltpu.VMEM((ROWS_PER_TILE,), jnp.float32),
        pltpu.SemaphoreType.DMA,
        pltpu.SemaphoreType.DMA,
        pltpu.SemaphoreType.DMA,
        pltpu.SemaphoreType.DMA,
        pltpu.SemaphoreType.DMA,
        pltpu.SemaphoreType.DMA,
        pltpu.SemaphoreType.DMA,
        pltpu.SemaphoreType.DMA,
        pltpu.VMEM_SHARED((N_PAD,), jnp.float32),
        pltpu.VMEM_SHARED((N_PAD,), jnp.float32),
    ],
    compiler_params=_sc_params,
)(_deg_body)


def _prep_body(emb_ref, w_ref, dout_ref, din_ref,
               c1_ref, p16_ref, qy_ref):
    x = emb_ref[...]
    ss = jnp.sum(x * x, axis=-1, keepdims=True)
    h = x / (jnp.sqrt(ss) + 1e-12)
    y0 = jnp.dot(h, w_ref[...], preferred_element_type=jnp.float32)
    din = din_ref[...]
    dout = dout_ref[...]
    rin = lax.rsqrt(jnp.where(din > 0, din, 1.0))
    rout = lax.rsqrt(jnp.where(dout > 0, dout, 1.0))
    c1_ref[...] = ALPHA * rin * y0
    p16_ref[...] = jnp.broadcast_to((1.0 - ALPHA) * rin * rout, (TBLK, 16))
    qy_ref[...] = jnp.concatenate([
        ALPHA * y0,
        jnp.broadcast_to((1.0 - ALPHA) * rout, (TBLK, 16)),
        jnp.zeros((TBLK, 48), jnp.float32),
    ], axis=1)


def _prep(emb_pad, w, dout, din):
    grid = (N_PAD // TBLK,)
    return pl.pallas_call(
        _prep_body,
        grid=grid,
        in_specs=[
            pl.BlockSpec((TBLK, HIDDEN), lambda i: (i, 0)),
            pl.BlockSpec((HIDDEN, NCLS), lambda i: (0, 0)),
            pl.BlockSpec((TBLK, 1), lambda i: (i, 0)),
            pl.BlockSpec((TBLK, 1), lambda i: (i, 0)),
        ],
        out_specs=[
            pl.BlockSpec((TBLK, NCLS), lambda i: (i, 0)),
            pl.BlockSpec((TBLK, 16), lambda i: (i, 0)),
            pl.BlockSpec((TBLK, 128), lambda i: (i, 0)),
        ],
        out_shape=[
            jax.ShapeDtypeStruct((N_PAD, NCLS), jnp.float32),
            jax.ShapeDtypeStruct((N_PAD, 16), jnp.float32),
            jax.ShapeDtypeStruct((N_PAD, 128), jnp.float32),
        ],
    )(emb_pad, w, dout, din)


def _main_body(row_hbm, col_hbm, c1_hbm, p16_hbm, qy_hbm,
               idx_hbm, b_hbm, out_hbm,
               rbufc, cbufc, rbufcB, cbufcB,
               gbuf0, gbuf1, gbuf2, gbuf3,
               abufA, abufB, ubufA, ubufB, c1bufA, c1bufB,
               p16bufA, p16bufB,
               idxbuf, qybuf, rdbuf, obuf, bbuf,
               sem0, sem1, sem2, sem3,
               ssem0, ssem1, ssem2, ssem3,
               usemSA, usemSB, usemCA, usemCB, usemPA, usemPB,
               isem0, isem1,
               U_sp, S_sp):
    s = lax.axis_index("s")
    rbase = s * ROWS_PER_TILE
    gb = (gbuf0, gbuf1, gbuf2, gbuf3)
    sm = (sem0, sem1, sem2, sem3)
    ssm = (ssem0, ssem1, ssem2, ssem3)
    ub = ((abufA, ubufA, c1bufA, p16bufA, usemSA, usemCA, usemPA),
          (abufB, ubufB, c1bufB, p16bufB, usemSB, usemCB, usemPB))

    def zb(i, carry):
        for cc in range(NCLS // 16):
            obuf[i, pl.ds(cc * 16, 16)] = jnp.zeros((16,), jnp.float32)
        return carry

    lax.fori_loop(0, B_PER_TILE, zb, 0)
    zsrc = obuf.at[pl.ds(0, UCH)]

    def init_chunk(t, carry):
        base = rbase + t * UCH
        pltpu.sync_copy(c1_hbm.at[pl.ds(base, UCH)], c1bufA)

        def rw(i, carry2):
            for cc in range(NCLS // 16):
                sl = pl.ds(cc * 16, 16)
                ubufA[i, sl] = c1bufA[i, sl] * (1.0 / ALPHA)
            return carry2

        lax.fori_loop(0, UCH, rw, 0)
        pltpu.sync_copy(ubufA, U_sp.at[pl.ds(base, UCH)])
        pltpu.sync_copy(zsrc, S_sp.at[pl.ds(base, UCH)])
        return carry

    lax.fori_loop(0, NUCH, init_chunk, 0)
    plsc.subcore_barrier()

    def scatter_phase():
        PAIR = 2 * GRP

        def irow(which, jj):
            rb, cb2 = (rbufc, cbufc) if jj < GRP else (rbufcB, cbufcB)
            return (rb.at[jj % GRP], cb2.at[jj % GRP])

        def g_issue(jj, blk_r, blk_c):
            del blk_r
            return pltpu.async_copy(
                U_sp.at[blk_c], gb[jj % NBUF], sm[jj % NBUF])

        pltpu.sync_copy(row_hbm.at[s, pl.ds(0, GRP)], rbufc)
        pltpu.sync_copy(col_hbm.at[s, pl.ds(0, GRP)], cbufc)
        for j in range(GLA):
            _, bc = irow(0, j)
            g_issue(j, None, bc)

        def pair_body(g, carry):
            base = g * PAIR
            pltpu.async_copy(
                row_hbm.at[s, pl.ds(base + GRP, GRP)], rbufcB, isem0)
            pltpu.async_copy(
                col_hbm.at[s, pl.ds(base + GRP, GRP)], cbufcB, isem1)
            for j in range(PAIR):
                cur = j % NBUF
                br, bc = irow(0, j)
                pltpu.make_async_copy(U_sp.at[bc], gb[cur], sm[cur]).wait()
                pltpu.async_copy(
                    gb[cur], S_sp.at[br], ssm[cur], add=True)
                if j == GRP - GLA:
                    pltpu.make_async_copy(
                        row_hbm.at[s, pl.ds(base + GRP, GRP)], rbufcB,
                        isem0).wait()
                    pltpu.make_async_copy(
                        col_hbm.at[s, pl.ds(base + GRP, GRP)], cbufcB,
                        isem1).wait()
                if j == GRP + GLA:
                    @pl.when(g < NPAIR - 1)
                    def _():
                        pltpu.async_copy(
                            row_hbm.at[s, pl.ds(base + PAIR, GRP)],
                            rbufc, isem0)
                        pltpu.async_copy(
                            col_hbm.at[s, pl.ds(base + PAIR, GRP)],
                            cbufc, isem1)
                nxt = j + GLA
                if nxt < PAIR:
                    nb = nxt % NBUF
                    if j < GLA:
                        @pl.when(g > 0)
                        def _():
                            pltpu.make_async_copy(
                                gb[nb], S_sp.at[br], ssm[nb]).wait()
                    else:
                        pltpu.make_async_copy(
                            gb[nb], S_sp.at[br], ssm[nb]).wait()
                    _, bc2 = irow(0, nxt)
                    g_issue(nxt, None, bc2)
                else:
                    @pl.when(g < NPAIR - 1)
                    def _():
                        if nxt == PAIR:
                            pltpu.make_async_copy(
                                row_hbm.at[s, pl.ds(base + PAIR, GRP)],
                                rbufc, isem0).wait()
                            pltpu.make_async_copy(
                                col_hbm.at[s, pl.ds(base + PAIR, GRP)],
                                cbufc, isem1).wait()
                        nb = nxt % NBUF
                        pltpu.make_async_copy(
                            gb[nb], S_sp.at[br], ssm[nb]).wait()
                        g_issue(nxt, None, cbufc.at[nxt % GRP])
            return carry

        lax.fori_loop(0, NPAIR, pair_body, 0)
        for k in range(NBUF):
            pltpu.make_async_copy(
                gb[k], S_sp.at[rbufc.at[k]], ssm[k]).wait()

    def _uissue(t, bs):
        ab, _, cb, pb, sS, sC, sP = ub[bs]
        base = rbase + t * UCH
        pltpu.async_copy(S_sp.at[pl.ds(base, UCH)], ab, sS)
        pltpu.async_copy(c1_hbm.at[pl.ds(base, UCH)], cb, sC)
        pltpu.async_copy(p16_hbm.at[pl.ds(base, UCH)], pb, sP)

    def _ucompute(t, bs):
        ab, uu, cb, pb, sS, sC, sP = ub[bs]
        base = rbase + t * UCH
        pltpu.make_async_copy(S_sp.at[pl.ds(base, UCH)], ab, sS).wait()
        pltpu.make_async_copy(c1_hbm.at[pl.ds(base, UCH)], cb, sC).wait()
        pltpu.make_async_copy(p16_hbm.at[pl.ds(base, UCH)], pb, sP).wait()

        def rw(i, carry2):
            p = pb[i]
            for cc in range(NCLS // 16):
                sl = pl.ds(cc * 16, 16)
                uu[i, sl] = p * ab[i, sl] + cb[i, sl]
            return carry2

        lax.fori_loop(0, UCH, rw, 0)
        pltpu.sync_copy(uu, U_sp.at[pl.ds(base, UCH)])
        pltpu.sync_copy(zsrc, S_sp.at[pl.ds(base, UCH)])

    def update_phase():
        _uissue(0, 0)

        def upair(g, carry):
            _uissue(2 * g + 1, 1)
            _ucompute(2 * g, 0)

            @pl.when(g < NUCH // 2 - 1)
            def _():
                _uissue(2 * g + 2, 0)

            _ucompute(2 * g + 1, 1)
            return carry

        lax.fori_loop(0, NUCH // 2, upair, 0)

    def kiter(k, carry):
        scatter_phase()
        plsc.subcore_barrier()
        update_phase()
        plsc.subcore_barrier()
        return carry

    lax.fori_loop(0, K_ITERS - 1, kiter, 0)
    scatter_phase()
    plsc.subcore_barrier()

    pltpu.sync_copy(b_hbm, bbuf)
    for h in range(B_PER_TILE // 16):
        ob = s * B_PER_TILE + h * 16
        pltpu.sync_copy(idx_hbm.at[pl.ds(ob, 16)], idxbuf)
        pltpu.sync_copy(qy_hbm.at[idxbuf], qybuf)
        pltpu.sync_copy(S_sp.at[idxbuf], rdbuf)

        def rbody(i, carry):
            q = qybuf[i, pl.ds(NCLS, 16)]
            for cc in range(NCLS // 16):
                sl = pl.ds(cc * 16, 16)
                obuf[h * 16 + i, sl] = (q * rdbuf[i, sl] + qybuf[i, sl]
                                        + bbuf[sl])
            return carry

        lax.fori_loop(0, 16, rbody, 0)
    pltpu.sync_copy(obuf, out_hbm.at[pl.ds(s * B_PER_TILE, B_PER_TILE)])


_main_kernel = functools.partial(
    pl.kernel,
    out_type=jax.ShapeDtypeStruct((BATCH, NCLS), jnp.float32),
    mesh=_mesh,
    scratch_types=[
        pltpu.VMEM((GRP, CHUNK), jnp.int32),
        pltpu.VMEM((GRP, CHUNK), jnp.int32),
        pltpu.VMEM((GRP, CHUNK), jnp.int32),
        pltpu.VMEM((GRP, CHUNK), jnp.int32),
        pltpu.VMEM((CHUNK, NCLS), jnp.float32),
        pltpu.VMEM((CHUNK, NCLS), jnp.float32),
        pltpu.VMEM((CHUNK, NCLS), jnp.float32),
        pltpu.VMEM((CHUNK, NCLS), jnp.float32),
        pltpu.VMEM((UCH, NCLS), jnp.float32),
        pltpu.VMEM((UCH, NCLS), jnp.float32),
        pltpu.VMEM((UCH, NCLS), jnp.float32),
        pltpu.VMEM((UCH, NCLS), jnp.float32),
        pltpu.VMEM((UCH, NCLS), jnp.float32),
        pltpu.VMEM((UCH, NCLS), jnp.float32),
        pltpu.VMEM((UCH, 16), jnp.float32),
        pltpu.VMEM((UCH, 16), jnp.float32),
        pltpu.VMEM((16,), jnp.int32),
        pltpu.VMEM((16, 128), jnp.float32),
        pltpu.VMEM((16, NCLS), jnp.float32),
        pltpu.VMEM((B_PER_TILE, NCLS), jnp.float32),
        pltpu.VMEM((NCLS,), jnp.float32),
        pltpu.SemaphoreType.DMA,
        pltpu.SemaphoreType.DMA,
        pltpu.SemaphoreType.DMA,
        pltpu.SemaphoreType.DMA,
        pltpu.SemaphoreType.DMA,
        pltpu.SemaphoreType.DMA,
        pltpu.SemaphoreType.DMA,
        pltpu.SemaphoreType.DMA,
        pltpu.SemaphoreType.DMA,
        pltpu.SemaphoreType.DMA,
        pltpu.SemaphoreType.DMA,
        pltpu.SemaphoreType.DMA,
        pltpu.SemaphoreType.DMA,
        pltpu.SemaphoreType.DMA,
        pltpu.SemaphoreType.DMA,
        pltpu.SemaphoreType.DMA,
        pltpu.VMEM_SHARED((N_PAD, NCLS), jnp.float32),
        pltpu.VMEM_SHARED((N_PAD, NCLS), jnp.float32),
    ],
    compiler_params=_sc_params,
)(_main_body)


def kernel(X, idx, edge_index, emb, W, b):
    del X
    emb_pad = jnp.pad(emb, ((0, N_PAD - N_NODES), (0, 0)))
    row = edge_index[0].astype(jnp.int32)
    col = edge_index[1].astype(jnp.int32)
    padv = jnp.full((E_PAD - N_EDGES,), N_NODES, jnp.int32)
    row3 = jnp.concatenate([row, padv]).reshape(NSUB, EC_PER_TILE, CHUNK)
    col3 = jnp.concatenate([col, padv]).reshape(NSUB, EC_PER_TILE, CHUNK)
    idx32 = idx.astype(jnp.int32)

    dout, din = _deg_kernel(row3, col3)
    c1, p16, qy = _prep(emb_pad, W, dout.reshape(N_PAD, 1),
                        din.reshape(N_PAD, 1))
    out = _main_kernel(row3, col3, c1, p16, qy, idx32, b)
    return out
